# Initial kernel scaffold; baseline (speedup 1.0000x reference)
import jax
import jax.numpy as jnp
from jax.experimental import pallas as pl


def kernel(local_x, voxel_x, label_hard, local_type, voxel_type, edge_index, params):
    raise NotImplementedError("write your pallas kernel here")


# speedup vs baseline: 9.0690x; 9.0690x over previous
import functools

import jax
import jax.numpy as jnp
from jax import lax
from jax.experimental import pallas as pl
from jax.experimental.pallas import tpu as pltpu
from jax.experimental.pallas import tpu_sc as plsc

N_LOCAL = 10000
N = 50000
E = 800000
NUM_TYPES = 8
HID = 64

NPAD = 50048
EPAD = 819200
CHUNK = 128
ZR = NPAD // 16
ZCH = 136
NZC = ZR // ZCH

BLK = 2000
GRID = N // BLK
LBLK = 2000
LGRID = N_LOCAL // LBLK

_f32 = jnp.float32


def _mesh():
    return plsc.VectorSubcoreMesh(core_axis_name="c", subcore_axis_name="s",
                                  num_cores=2, num_subcores=16)


@functools.partial(
    pl.kernel,
    out_type=(jax.ShapeDtypeStruct((NPAD,), _f32),
              jax.ShapeDtypeStruct((NPAD,), _f32)),
    mesh=_mesh(),
    compiler_params=pltpu.CompilerParams(use_tc_tiling_on_sc=False),
    scratch_types=[
        pltpu.VMEM_SHARED((NPAD,), _f32),
        pltpu.VMEM((ZCH,), _f32),
        pltpu.VMEM((CHUNK,), _f32),
        pltpu.VMEM((CHUNK,), jnp.int32),
    ],
)
def _deg_sc(dst_hbm, ones_hbm, zeros_hbm, out0_hbm, out1_hbm,
            acc, zbuf, onesb, dstb):
    c = lax.axis_index("c")
    s = lax.axis_index("s")
    pltpu.sync_copy(zeros_hbm, zbuf)

    def zbody(k, carry):
        pltpu.sync_copy(zbuf, acc.at[pl.ds(s * ZR + k * ZCH, ZCH)])
        return carry

    lax.fori_loop(0, NZC, zbody, 0)
    plsc.subcore_barrier()
    pltpu.sync_copy(ones_hbm, onesb)
    wbase = (c * 16 + s) * (EPAD // 32)

    def body(j, carry):
        pltpu.sync_copy(dst_hbm.at[pl.ds(wbase + j * CHUNK, CHUNK)], dstb)
        pltpu.sync_copy(onesb, acc.at[dstb], add=True)
        return carry

    lax.fori_loop(0, EPAD // 32 // CHUNK, body, 0)
    plsc.subcore_barrier()

    def copy_out(out_hbm):
        def cbody(k, carry):
            pltpu.sync_copy(acc.at[pl.ds(s * ZR + k * ZCH, ZCH)], zbuf)
            pltpu.sync_copy(zbuf, out_hbm.at[pl.ds(s * ZR + k * ZCH, ZCH)])
            return carry

        lax.fori_loop(0, NZC, cbody, 0)

    @pl.when(c == 0)
    def _():
        copy_out(out0_hbm)

    @pl.when(c == 1)
    def _():
        copy_out(out1_hbm)


def _make_agg_cs(Cc):
    nchunks = EPAD // 16 // CHUNK

    @functools.partial(
        pl.kernel,
        out_type=jax.ShapeDtypeStruct((2, NPAD, Cc), _f32),
        mesh=_mesh(),
        compiler_params=pltpu.CompilerParams(use_tc_tiling_on_sc=False),
        scratch_types=[
            pltpu.VMEM_SHARED((NPAD, Cc), _f32),
            pltpu.VMEM((ZCH, Cc), _f32),
            pltpu.VMEM((CHUNK,), jnp.int32),
            pltpu.VMEM((CHUNK,), jnp.int32),
            pltpu.VMEM((CHUNK, Cc), _f32),
            pltpu.SemaphoreType.DMA,
        ],
    )
    def k(y_lo, y_hi, src_hbm, dst_hbm, zeros_hbm, out_hbm,
          acc, zbuf, srcb, dstb, rows, sem):
        c = lax.axis_index("c")
        s = lax.axis_index("s")
        pltpu.sync_copy(zeros_hbm, zbuf)

        def zbody(k, carry):
            pltpu.sync_copy(zbuf, acc.at[pl.ds(s * ZR + k * ZCH, ZCH), :])
            return carry

        lax.fori_loop(0, NZC, zbody, 0)
        plsc.subcore_barrier()
        sbase = s * (EPAD // 16)

        def body(j, carry):
            base = sbase + j * CHUNK
            pltpu.sync_copy(src_hbm.at[pl.ds(base, CHUNK)], srcb)
            pltpu.sync_copy(dst_hbm.at[pl.ds(base, CHUNK)], dstb)

            @pl.when(c == 0)
            def _():
                pltpu.async_copy(y_lo.at[srcb], rows, sem).wait()

            @pl.when(c == 1)
            def _():
                pltpu.async_copy(y_hi.at[srcb], rows, sem).wait()

            pltpu.sync_copy(rows, acc.at[dstb], add=True)
            return carry

        lax.fori_loop(0, nchunks, body, 0)
        plsc.subcore_barrier()

        def cbody(k, carry):
            pltpu.sync_copy(acc.at[pl.ds(s * ZR + k * ZCH, ZCH), :], zbuf)
            pltpu.sync_copy(zbuf, out_hbm.at[c, pl.ds(s * ZR + k * ZCH, ZCH), :])
            return carry

        lax.fori_loop(0, NZC, cbody, 0)

    return k


def _make_agg_es(C):
    nchunks = EPAD // 32 // CHUNK

    @functools.partial(
        pl.kernel,
        out_type=jax.ShapeDtypeStruct((2, NPAD, C), _f32),
        mesh=_mesh(),
        compiler_params=pltpu.CompilerParams(use_tc_tiling_on_sc=False),
        scratch_types=[
            pltpu.VMEM_SHARED((NPAD, C), _f32),
            pltpu.VMEM((ZCH, C), _f32),
            pltpu.VMEM((CHUNK,), jnp.int32),
            pltpu.VMEM((CHUNK,), jnp.int32),
            pltpu.VMEM((CHUNK, C), _f32),
            pltpu.SemaphoreType.DMA,
        ],
    )
    def k(y_hbm, src_hbm, dst_hbm, zeros_hbm, out_hbm,
          acc, zbuf, srcb, dstb, rows, sem):
        c = lax.axis_index("c")
        s = lax.axis_index("s")
        pltpu.sync_copy(zeros_hbm, zbuf)

        def zbody(k, carry):
            pltpu.sync_copy(zbuf, acc.at[pl.ds(s * ZR + k * ZCH, ZCH), :])
            return carry

        lax.fori_loop(0, NZC, zbody, 0)
        plsc.subcore_barrier()
        wbase = (c * 16 + s) * (EPAD // 32)

        def body(j, carry):
            base = wbase + j * CHUNK
            pltpu.sync_copy(src_hbm.at[pl.ds(base, CHUNK)], srcb)
            pltpu.sync_copy(dst_hbm.at[pl.ds(base, CHUNK)], dstb)
            pltpu.async_copy(y_hbm.at[srcb], rows, sem).wait()
            pltpu.sync_copy(rows, acc.at[dstb], add=True)
            return carry

        lax.fori_loop(0, nchunks, body, 0)
        plsc.subcore_barrier()

        def cbody(k, carry):
            pltpu.sync_copy(acc.at[pl.ds(s * ZR + k * ZCH, ZCH), :], zbuf)
            pltpu.sync_copy(zbuf, out_hbm.at[c, pl.ds(s * ZR + k * ZCH, ZCH), :])
            return carry

        lax.fori_loop(0, NZC, cbody, 0)

    return k


_agg_cs16 = _make_agg_cs(16)
_agg_cs32 = _make_agg_cs(32)
_agg_es16 = _make_agg_es(16)


def _k1_body(lt_ref, x_ref, w1a_ref, t8_ref, acc_ref, cnt_ref):
    i = pl.program_id(0)

    @pl.when(i == 0)
    def _():
        acc_ref[...] = jnp.zeros_like(acc_ref)
        cnt_ref[...] = jnp.zeros_like(cnt_ref)

    lt = lt_ref[0, 0, :]
    oh = (lt[:, None] == lax.broadcasted_iota(jnp.int32, (LBLK, NUM_TYPES), 1)
          ).astype(_f32)
    acc_ref[...] += lax.dot_general(oh, x_ref[...], (((0,), (0,)), ((), ())),
                                    preferred_element_type=_f32)
    cnt_ref[...] += jnp.sum(oh, axis=0, keepdims=True)

    @pl.when(i == pl.num_programs(0) - 1)
    def _():
        cnt = cnt_ref[0, :]
        tm = acc_ref[...] / jnp.maximum(cnt, 1.0)[:, None]
        tm = jnp.where(cnt[:, None] > 0, tm, 0.0)
        t8_ref[...] = jnp.dot(tm, w1a_ref[...], preferred_element_type=_f32)


def _k1(lt3, local_x, w1a):
    return pl.pallas_call(
        _k1_body,
        grid=(LGRID,),
        in_specs=[
            pl.BlockSpec((1, 1, LBLK), lambda i: (i, 0, 0)),
            pl.BlockSpec((LBLK, 128), lambda i: (i, 0)),
            pl.BlockSpec((128, HID), lambda i: (0, 0)),
        ],
        out_specs=pl.BlockSpec((NUM_TYPES, HID), lambda i: (0, 0)),
        out_shape=jax.ShapeDtypeStruct((NUM_TYPES, HID), _f32),
        scratch_shapes=[
            pltpu.VMEM((NUM_TYPES, 128), _f32),
            pltpu.VMEM((1, NUM_TYPES), _f32),
        ],
    )(lt3, local_x, w1a)


def _k2_body(vt_ref, vx_ref, lb_ref, degT_ref, t8_ref, w1b_ref, w1c_ref,
             b1_ref, w2_ref, b2_ref, wc0_ref, ylo_ref, yhi_ref):
    deg = degT_ref[:, 0] + degT_ref[:, 1] + 1.0
    dinv = lax.rsqrt(deg)
    vt = vt_ref[0, 0, :]
    oh = (vt[:, None] == lax.broadcasted_iota(jnp.int32, (BLK, NUM_TYPES), 1)
          ).astype(_f32)
    m = jnp.dot(oh, t8_ref[...], preferred_element_type=_f32)
    h = m + jnp.dot(vx_ref[...], w1b_ref[...], preferred_element_type=_f32)
    h = h + jnp.dot(lb_ref[...], w1c_ref[...], preferred_element_type=_f32)
    h = jnp.maximum(h + b1_ref[...], 0.0)
    x = jnp.maximum(jnp.dot(h, w2_ref[...], preferred_element_type=_f32)
                    + b2_ref[...], 0.0)
    y = jnp.dot(x, wc0_ref[...], preferred_element_type=_f32) * dinv[:, None]
    ylo_ref[...] = y[:, :16]
    yhi_ref[...] = y[:, 16:]


def _k2(vt3, voxel_x, lb, degT, t8, w1b, w1c, b1, w2, b2, wc0):
    return pl.pallas_call(
        _k2_body,
        grid=(GRID,),
        in_specs=[
            pl.BlockSpec((1, 1, BLK), lambda i: (i, 0, 0)),
            pl.BlockSpec((BLK, 128), lambda i: (i, 0)),
            pl.BlockSpec((BLK, 16), lambda i: (i, 0)),
            pl.BlockSpec((BLK, 2), lambda i: (i, 0)),
            pl.BlockSpec((NUM_TYPES, HID), lambda i: (0, 0)),
            pl.BlockSpec((128, HID), lambda i: (0, 0)),
            pl.BlockSpec((16, HID), lambda i: (0, 0)),
            pl.BlockSpec((1, HID), lambda i: (0, 0)),
            pl.BlockSpec((HID, HID), lambda i: (0, 0)),
            pl.BlockSpec((1, HID), lambda i: (0, 0)),
            pl.BlockSpec((HID, 32), lambda i: (0, 0)),
        ],
        out_specs=[
            pl.BlockSpec((BLK, 16), lambda i: (i, 0)),
            pl.BlockSpec((BLK, 16), lambda i: (i, 0)),
        ],
        out_shape=[
            jax.ShapeDtypeStruct((N, 16), _f32),
            jax.ShapeDtypeStruct((N, 16), _f32),
        ],
    )(vt3, voxel_x, lb, degT, t8, w1b, w1c, b1, w2, b2, wc0)


def _make_k3(C, concat):
    Cc = C // 2 if concat else C

    def body(agg_ref, ylo_ref, yhi_ref, degT_ref, b_ref, h_ref, st_ref):
        i = pl.program_id(0)

        @pl.when(i == 0)
        def _():
            st_ref[...] = jnp.zeros_like(st_ref)

        deg = degT_ref[:, 0] + degT_ref[:, 1] + 1.0
        dinv = lax.rsqrt(deg)
        if concat:
            acc = jnp.concatenate([agg_ref[0], agg_ref[1]], axis=-1)
            y = jnp.concatenate([ylo_ref[...], yhi_ref[...]], axis=-1)
        else:
            acc = agg_ref[0] + agg_ref[1]
            y = ylo_ref[...]
        h = dinv[:, None] * (acc + y) + b_ref[...]
        h_ref[...] = h
        st_ref[0:1, :] += jnp.sum(h, axis=0, keepdims=True)
        st_ref[1:2, :] += jnp.sum(h * h, axis=0, keepdims=True)

    def run(agg, ylo, yhi, degT, b):
        ins = [agg, ylo] + ([yhi] if concat else [ylo]) + [degT, b]
        return pl.pallas_call(
            body,
            grid=(GRID,),
            in_specs=[
                pl.BlockSpec((2, BLK, Cc), lambda i: (0, i, 0)),
                pl.BlockSpec((BLK, Cc), lambda i: (i, 0)),
                pl.BlockSpec((BLK, Cc), lambda i: (i, 0)),
                pl.BlockSpec((BLK, 2), lambda i: (i, 0)),
                pl.BlockSpec((1, C), lambda i: (0, 0)),
            ],
            out_specs=[
                pl.BlockSpec((BLK, C), lambda i: (i, 0)),
                pl.BlockSpec((8, C), lambda i: (0, 0)),
            ],
            out_shape=[
                jax.ShapeDtypeStruct((N, C), _f32),
                jax.ShapeDtypeStruct((8, C), _f32),
            ],
        )(*ins)

    return run


_k3_c32 = _make_k3(32, True)
_k3_a16 = _make_k3(16, False)
_k3_c64 = _make_k3(64, True)


def _norm_x(h, st, gw, ga, gb):
    mean = st[0:1, :] * (1.0 / N)
    ex2 = st[1:2, :] * (1.0 / N)
    var = ex2 - ga * (2.0 - ga) * mean * mean
    xc = h - ga * mean
    xn = gw * xc * lax.rsqrt(var + 1e-5) + gb
    return jnp.maximum(xn, 0.0)


def _make_k4(C, Cn, split):
    def body(h_ref, st_ref, gw_ref, gb_ref, ga_ref, degT_ref, w_ref, *outs):
        deg = degT_ref[:, 0] + degT_ref[:, 1] + 1.0
        dinv = lax.rsqrt(deg)
        xn = _norm_x(h_ref[...], st_ref[...], gw_ref[...], ga_ref[...],
                     gb_ref[...])
        y = jnp.dot(xn, w_ref[...], preferred_element_type=_f32) * dinv[:, None]
        if split:
            outs[0][...] = y[:, :Cn // 2]
            outs[1][...] = y[:, Cn // 2:]
        else:
            outs[0][...] = y

    n_out = 2 if split else 1
    Co = Cn // 2 if split else Cn

    def run(h, st, gw, gb, ga, degT, w):
        return pl.pallas_call(
            body,
            grid=(GRID,),
            in_specs=[
                pl.BlockSpec((BLK, C), lambda i: (i, 0)),
                pl.BlockSpec((8, C), lambda i: (0, 0)),
                pl.BlockSpec((1, C), lambda i: (0, 0)),
                pl.BlockSpec((1, C), lambda i: (0, 0)),
                pl.BlockSpec((1, C), lambda i: (0, 0)),
                pl.BlockSpec((BLK, 2), lambda i: (i, 0)),
                pl.BlockSpec((C, Cn), lambda i: (0, 0)),
            ],
            out_specs=[pl.BlockSpec((BLK, Co), lambda i: (i, 0))] * n_out,
            out_shape=[jax.ShapeDtypeStruct((N, Co), _f32)] * n_out,
        )(h, st, gw, gb, ga, degT, w)

    return run


_k4_l0 = _make_k4(32, 16, False)
_k4_l1 = _make_k4(16, 32, True)
_k4_l2 = _make_k4(32, 64, True)


def _k4_dec_body(h_ref, st_ref, gw_ref, gb_ref, ga_ref,
                 d0w_ref, d0b_ref, d1w_ref, d1b_ref, d2w_ref, d2b_ref,
                 d3w_ref, d3b_ref, out_ref):
    xn = _norm_x(h_ref[...], st_ref[...], gw_ref[...], ga_ref[...], gb_ref[...])
    d = jnp.maximum(jnp.dot(xn, d0w_ref[...], preferred_element_type=_f32)
                    + d0b_ref[...], 0.0)
    d = jnp.maximum(jnp.dot(d, d1w_ref[...], preferred_element_type=_f32)
                    + d1b_ref[...], 0.0)
    d = jnp.maximum(jnp.dot(d, d2w_ref[...], preferred_element_type=_f32)
                    + d2b_ref[...], 0.0)
    z = jnp.dot(d, d3w_ref[...], preferred_element_type=_f32) + d3b_ref[...]
    out_ref[...] = 1.0 / (1.0 + jnp.exp(-z))


def _k4_dec(h, st, gw, gb, ga, dws):
    return pl.pallas_call(
        _k4_dec_body,
        grid=(GRID,),
        in_specs=[
            pl.BlockSpec((BLK, HID), lambda i: (i, 0)),
            pl.BlockSpec((8, HID), lambda i: (0, 0)),
            pl.BlockSpec((1, HID), lambda i: (0, 0)),
            pl.BlockSpec((1, HID), lambda i: (0, 0)),
            pl.BlockSpec((1, HID), lambda i: (0, 0)),
            pl.BlockSpec((HID, 32), lambda i: (0, 0)),
            pl.BlockSpec((1, 32), lambda i: (0, 0)),
            pl.BlockSpec((32, 16), lambda i: (0, 0)),
            pl.BlockSpec((1, 16), lambda i: (0, 0)),
            pl.BlockSpec((16, 8), lambda i: (0, 0)),
            pl.BlockSpec((1, 8), lambda i: (0, 0)),
            pl.BlockSpec((8, 1), lambda i: (0, 0)),
            pl.BlockSpec((1, 1), lambda i: (0, 0)),
        ],
        out_specs=pl.BlockSpec((BLK, 1), lambda i: (i, 0)),
        out_shape=jax.ShapeDtypeStruct((N, 1), _f32),
    )(h, st, gw, gb, ga, *dws)


def kernel(local_x, voxel_x, label_hard, local_type, voxel_type, edge_index,
           params):
    p = params
    src = edge_index[0].astype(jnp.int32)
    dst = edge_index[1].astype(jnp.int32)
    npad = EPAD - E
    srcp = jnp.concatenate([src, jnp.zeros((npad,), jnp.int32)])
    dstp = jnp.concatenate([dst, jnp.full((npad,), N, jnp.int32)])
    lt3 = local_type.astype(jnp.int32).reshape(LGRID, 1, LBLK)
    vt3 = voxel_type.astype(jnp.int32).reshape(GRID, 1, BLK)
    lb = label_hard[0]

    ones_c = jnp.ones((CHUNK,), _f32)
    zeros_1 = jnp.zeros((ZCH,), _f32)
    zeros_16 = jnp.zeros((ZCH, 16), _f32)
    zeros_32 = jnp.zeros((ZCH, 32), _f32)

    deg0, deg1 = _deg_sc(dstp, ones_c, zeros_1)
    degT = jnp.stack([deg0, deg1], axis=1)

    t8 = _k1(lt3, local_x, p['mlp_W1'][:128])
    ylo0, yhi0 = _k2(vt3, voxel_x, lb, degT, t8,
                     p['mlp_W1'][128:256], p['mlp_W1'][256:],
                     p['mlp_b1'].reshape(1, HID),
                     p['mlp_W2'], p['mlp_b2'].reshape(1, HID),
                     p['conv0_W'])

    agg0 = _agg_cs16(ylo0, yhi0, srcp, dstp, zeros_16)
    h0, st0 = _k3_c32(agg0, ylo0, yhi0, degT, p['conv0_b'].reshape(1, 32))
    y1 = _k4_l0(h0, st0, p['gn0_w'].reshape(1, 32), p['gn0_b'].reshape(1, 32),
                p['gn0_a'].reshape(1, 32), degT, p['conv1_W'])[0]

    agg1 = _agg_es16(y1, srcp, dstp, zeros_16)
    h1, st1 = _k3_a16(agg1, y1, None, degT, p['conv1_b'].reshape(1, 16))
    ylo2, yhi2 = _k4_l1(h1, st1, p['gn1_w'].reshape(1, 16),
                        p['gn1_b'].reshape(1, 16), p['gn1_a'].reshape(1, 16),
                        degT, p['conv2_W'])

    agg2 = _agg_cs16(ylo2, yhi2, srcp, dstp, zeros_16)
    h2, st2 = _k3_c32(agg2, ylo2, yhi2, degT, p['conv2_b'].reshape(1, 32))
    ylo3, yhi3 = _k4_l2(h2, st2, p['gn2_w'].reshape(1, 32),
                        p['gn2_b'].reshape(1, 32), p['gn2_a'].reshape(1, 32),
                        degT, p['conv3_W'])

    agg3 = _agg_cs32(ylo3, yhi3, srcp, dstp, zeros_32)
    h3, st3 = _k3_c64(agg3, ylo3, yhi3, degT, p['conv3_b'].reshape(1, 64))
    out = _k4_dec(h3, st3, p['gn3_w'].reshape(1, HID),
                  p['gn3_b'].reshape(1, HID), p['gn3_a'].reshape(1, HID),
                  [p['dec0_W'], p['dec0_b'].reshape(1, 32),
                   p['dec1_W'], p['dec1_b'].reshape(1, 16),
                   p['dec2_W'], p['dec2_b'].reshape(1, 8),
                   p['dec3_W'], p['dec3_b'].reshape(1, 1)])
    return out


# speedup vs baseline: 15.9037x; 1.7536x over previous
import functools

import jax
import jax.numpy as jnp
from jax import lax
from jax.experimental import pallas as pl
from jax.experimental.pallas import tpu as pltpu
from jax.experimental.pallas import tpu_sc as plsc

N_LOCAL = 10000
N = 50000
E = 800000
NUM_TYPES = 8
HID = 64

NPAD = 50048
EPAD = 819200
CHUNK = 128
ZR = NPAD // 16
ZCH = 136
NZC = ZR // ZCH
SLAB = 40
HALF = 5

BLK = 2000
GRID = N // BLK
LBLK = 2000
LGRID = N_LOCAL // LBLK

_f32 = jnp.float32


def _mesh():
    return plsc.VectorSubcoreMesh(core_axis_name="c", subcore_axis_name="s",
                                  num_cores=2, num_subcores=16)


@functools.partial(
    pl.kernel,
    out_type=(jax.ShapeDtypeStruct((NPAD,), _f32),
              jax.ShapeDtypeStruct((NPAD,), _f32)),
    mesh=_mesh(),
    compiler_params=pltpu.CompilerParams(use_tc_tiling_on_sc=False),
    scratch_types=[
        pltpu.VMEM_SHARED((NPAD,), _f32),
        pltpu.VMEM((ZCH,), _f32),
        pltpu.VMEM((CHUNK,), _f32),
        pltpu.VMEM((2, CHUNK), jnp.int32),
        pltpu.SemaphoreType.DMA,
        pltpu.SemaphoreType.DMA,
        pltpu.SemaphoreType.DMA,
    ],
)
def _deg_sc(dst_hbm, ones_hbm, zeros_hbm, out0_hbm, out1_hbm,
            acc, zbuf, onesb, dstb, zsem, isem, ssem):
    c = lax.axis_index("c")
    s = lax.axis_index("s")
    pltpu.sync_copy(zeros_hbm, zbuf)
    zds = [pltpu.async_copy(zbuf, acc.at[pl.ds(s * ZR + k * ZCH, ZCH)], zsem)
           for k in range(NZC)]
    for d in zds:
        d.wait()
    plsc.subcore_barrier()
    pltpu.sync_copy(ones_hbm, onesb)
    wbase = (c * 16 + s) * (EPAD // 32)
    nch = EPAD // 32 // CHUNK

    def body(j, carry):
        par = j % 2
        base = wbase + j * CHUNK

        @pl.when(j >= 2)
        def _():
            pltpu.make_async_copy(zeros_hbm.at[pl.ds(0, CHUNK)],
                                  onesb, ssem).wait()

        pltpu.async_copy(dst_hbm.at[pl.ds(base, CHUNK)], dstb.at[par],
                         isem).wait()
        pltpu.async_copy(onesb, acc.at[dstb.at[par]], ssem, add=True)
        return carry

    lax.fori_loop(0, nch, body, 0)
    for _ in range(2):
        pltpu.make_async_copy(zeros_hbm.at[pl.ds(0, CHUNK)], onesb, ssem).wait()
    plsc.subcore_barrier()

    def copy_out(out_hbm):
        def cbody(k, carry):
            pltpu.sync_copy(acc.at[pl.ds(s * ZR + k * ZCH, ZCH)], zbuf)
            pltpu.sync_copy(zbuf, out_hbm.at[pl.ds(s * ZR + k * ZCH, ZCH)])
            return carry

        lax.fori_loop(0, NZC, cbody, 0)

    @pl.when(c == 0)
    def _():
        copy_out(out0_hbm)

    @pl.when(c == 1)
    def _():
        copy_out(out1_hbm)


def _make_agg(W, edge_split):
    sub_chunks = (EPAD // 32 if edge_split else EPAD // 16) // CHUNK
    nslab = sub_chunks // SLAB

    @functools.partial(
        pl.kernel,
        out_type=jax.ShapeDtypeStruct((2, NPAD, W), _f32),
        mesh=_mesh(),
        compiler_params=pltpu.CompilerParams(use_tc_tiling_on_sc=False),
        scratch_types=[
            pltpu.VMEM_SHARED((NPAD, W), _f32),
            pltpu.VMEM((ZCH, W), _f32),
            pltpu.VMEM((2, ZCH, W), _f32),
            pltpu.VMEM((SLAB, CHUNK), jnp.int32),
            pltpu.VMEM((2 * SLAB, CHUNK), jnp.int32),
            pltpu.VMEM((2 * HALF, CHUNK, W), _f32),
            pltpu.SemaphoreType.DMA,
            pltpu.SemaphoreType.DMA,
            pltpu.SemaphoreType.DMA,
            pltpu.SemaphoreType.DMA,
            pltpu.SemaphoreType.DMA,
            pltpu.SemaphoreType.DMA,
            pltpu.SemaphoreType.DMA,
            pltpu.SemaphoreType.DMA,
        ],
    )
    def k(ylo, yhi, src2d, dst2d, zeros_hbm, out_hbm,
          acc, zbuf, cbuf, src_slab, dst_slab, rows,
          gsA, gsB, ssA, ssB, zsem, rsem, ws0, ws1):
        c = lax.axis_index("c")
        s = lax.axis_index("s")
        gsems = (gsA, gsB)
        ssems = (ssA, ssB)
        wsems = (ws0, ws1)
        drain_src = ylo.at[pl.ds(0, CHUNK), :]

        pltpu.sync_copy(zeros_hbm, zbuf)
        zds = [pltpu.async_copy(zbuf, acc.at[pl.ds(s * ZR + k * ZCH, ZCH), :],
                                zsem)
               for k in range(NZC)]
        for d in zds:
            d.wait()
        plsc.subcore_barrier()

        if edge_split:
            base_row = (c * 16 + s) * sub_chunks
        else:
            base_row = s * sub_chunks

        def gather_half(tab, h, off):
            gds = [pltpu.async_copy(tab.at[src_slab.at[off + b]],
                                    rows.at[h * HALF + b], gsems[h])
                   for b in range(HALF)]
            for d in gds:
                d.wait()

        def slab_body(sl, carry):
            srow = base_row + sl * SLAB
            spar = (sl % 2) * SLAB
            pltpu.sync_copy(src2d.at[pl.ds(srow, SLAB), :], src_slab)
            pltpu.sync_copy(dst2d.at[pl.ds(srow, SLAB), :],
                            dst_slab.at[pl.ds(spar, SLAB), :])

            def grp_body(g, carry2):
                primed = (sl > 0) | (g > 0)
                for h in range(2):
                    off = g * (2 * HALF) + h * HALF
                    doff = spar + off

                    @pl.when(primed)
                    def _(h=h):
                        for b in range(HALF):
                            pltpu.make_async_copy(
                                drain_src, rows.at[h * HALF + b],
                                ssems[h]).wait()

                    if edge_split:
                        gather_half(ylo, h, off)
                    else:
                        @pl.when(c == 0)
                        def _(h=h, off=off):
                            gather_half(ylo, h, off)

                        @pl.when(c == 1)
                        def _(h=h, off=off):
                            gather_half(yhi, h, off)

                    for b in range(HALF):
                        pltpu.async_copy(rows.at[h * HALF + b],
                                         acc.at[dst_slab.at[doff + b]],
                                         ssems[h], add=True)
                return carry2

            lax.fori_loop(0, SLAB // (2 * HALF), grp_body, 0)
            return carry

        lax.fori_loop(0, nslab, slab_body, 0)
        for h in range(2):
            for b in range(HALF):
                pltpu.make_async_copy(drain_src, rows.at[h * HALF + b],
                                      ssems[h]).wait()
        plsc.subcore_barrier()

        wr = [None, None]
        for kk in range(NZC):
            par = kk % 2
            if wr[par] is not None:
                wr[par].wait()
            pltpu.async_copy(acc.at[pl.ds(s * ZR + kk * ZCH, ZCH), :],
                             cbuf.at[par], rsem).wait()
            wr[par] = pltpu.async_copy(
                cbuf.at[par], out_hbm.at[c, pl.ds(s * ZR + kk * ZCH, ZCH), :],
                wsems[par])
        wr[0].wait()
        wr[1].wait()

    return k


_agg_es16 = _make_agg(16, True)
_agg_cs16 = _make_agg(16, False)


def _k1_body(lt_ref, x_ref, w1a_ref, t8_ref, acc_ref, cnt_ref):
    i = pl.program_id(0)

    @pl.when(i == 0)
    def _():
        acc_ref[...] = jnp.zeros_like(acc_ref)
        cnt_ref[...] = jnp.zeros_like(cnt_ref)

    lt = lt_ref[0, 0, :]
    oh = (lt[:, None] == lax.broadcasted_iota(jnp.int32, (LBLK, NUM_TYPES), 1)
          ).astype(_f32)
    acc_ref[...] += lax.dot_general(oh, x_ref[...], (((0,), (0,)), ((), ())),
                                    preferred_element_type=_f32)
    cnt_ref[...] += jnp.sum(oh, axis=0, keepdims=True)

    @pl.when(i == pl.num_programs(0) - 1)
    def _():
        cnt = cnt_ref[0, :]
        tm = acc_ref[...] / jnp.maximum(cnt, 1.0)[:, None]
        tm = jnp.where(cnt[:, None] > 0, tm, 0.0)
        t8_ref[...] = jnp.dot(tm, w1a_ref[...], preferred_element_type=_f32)


def _k1(lt3, local_x, w1a):
    return pl.pallas_call(
        _k1_body,
        grid=(LGRID,),
        in_specs=[
            pl.BlockSpec((1, 1, LBLK), lambda i: (i, 0, 0)),
            pl.BlockSpec((LBLK, 128), lambda i: (i, 0)),
            pl.BlockSpec((128, HID), lambda i: (0, 0)),
        ],
        out_specs=pl.BlockSpec((NUM_TYPES, HID), lambda i: (0, 0)),
        out_shape=jax.ShapeDtypeStruct((NUM_TYPES, HID), _f32),
        scratch_shapes=[
            pltpu.VMEM((NUM_TYPES, 128), _f32),
            pltpu.VMEM((1, NUM_TYPES), _f32),
        ],
    )(lt3, local_x, w1a)


def _k2_body(vt_ref, vx_ref, lb_ref, degT_ref, t8_ref, w1b_ref, w1c_ref,
             b1_ref, w2_ref, b2_ref, wc0_ref, y_ref, y2_ref):
    deg = degT_ref[:, 0] + degT_ref[:, 1] + 1.0
    dinv = lax.rsqrt(deg)
    vt = vt_ref[0, 0, :]
    oh = (vt[:, None] == lax.broadcasted_iota(jnp.int32, (BLK, NUM_TYPES), 1)
          ).astype(_f32)
    m = jnp.dot(oh, t8_ref[...], preferred_element_type=_f32)
    h = m + jnp.dot(vx_ref[...], w1b_ref[...], preferred_element_type=_f32)
    h = h + jnp.dot(lb_ref[...], w1c_ref[...], preferred_element_type=_f32)
    h = jnp.maximum(h + b1_ref[...], 0.0)
    x = jnp.maximum(jnp.dot(h, w2_ref[...], preferred_element_type=_f32)
                    + b2_ref[...], 0.0)
    y = (jnp.dot(x, wc0_ref[...], preferred_element_type=_f32)
         * dinv[:, None])
    y_ref[...] = y[:, :16]
    y2_ref[...] = y[:, 16:]


def _k2(vt3, voxel_x, lb, degT, t8, w1b, w1c, b1, w2, b2, wc0):
    return pl.pallas_call(
        _k2_body,
        grid=(GRID,),
        in_specs=[
            pl.BlockSpec((1, 1, BLK), lambda i: (i, 0, 0)),
            pl.BlockSpec((BLK, 128), lambda i: (i, 0)),
            pl.BlockSpec((BLK, 16), lambda i: (i, 0)),
            pl.BlockSpec((BLK, 2), lambda i: (i, 0)),
            pl.BlockSpec((NUM_TYPES, HID), lambda i: (0, 0)),
            pl.BlockSpec((128, HID), lambda i: (0, 0)),
            pl.BlockSpec((16, HID), lambda i: (0, 0)),
            pl.BlockSpec((1, HID), lambda i: (0, 0)),
            pl.BlockSpec((HID, HID), lambda i: (0, 0)),
            pl.BlockSpec((1, HID), lambda i: (0, 0)),
            pl.BlockSpec((HID, 32), lambda i: (0, 0)),
        ],
        out_specs=[pl.BlockSpec((BLK, 16), lambda i: (i, 0))] * 2,
        out_shape=[jax.ShapeDtypeStruct((N, 16), _f32)] * 2,
    )(vt3, voxel_x, lb, degT, t8, w1b, w1c, b1, w2, b2, wc0)


def _make_k3(C, concat):
    Cc = C // 2 if concat else C

    def body(agg_ref, ylo_ref, yhi_ref, degT_ref, b_ref, h_ref, st_ref):
        i = pl.program_id(0)

        @pl.when(i == 0)
        def _():
            st_ref[...] = jnp.zeros_like(st_ref)

        deg = degT_ref[:, 0] + degT_ref[:, 1] + 1.0
        dinv = lax.rsqrt(deg)
        if concat:
            acc = jnp.concatenate([agg_ref[0], agg_ref[1]], axis=-1)
            y = jnp.concatenate([ylo_ref[...], yhi_ref[...]], axis=-1)
        else:
            acc = agg_ref[0] + agg_ref[1]
            y = ylo_ref[...]
        h = dinv[:, None] * (acc + y) + b_ref[...]
        h_ref[...] = h
        st_ref[0:1, :] += jnp.sum(h, axis=0, keepdims=True)
        st_ref[1:2, :] += jnp.sum(h * h, axis=0, keepdims=True)

    def run(agg, ylo, yhi, degT, b):
        ins = [agg, ylo] + ([yhi] if concat else [ylo]) + [degT, b]
        return pl.pallas_call(
            body,
            grid=(GRID,),
            in_specs=[
                pl.BlockSpec((2, BLK, Cc), lambda i: (0, i, 0)),
                pl.BlockSpec((BLK, Cc), lambda i: (i, 0)),
                pl.BlockSpec((BLK, Cc), lambda i: (i, 0)),
                pl.BlockSpec((BLK, 2), lambda i: (i, 0)),
                pl.BlockSpec((1, C), lambda i: (0, 0)),
            ],
            out_specs=[
                pl.BlockSpec((BLK, C), lambda i: (i, 0)),
                pl.BlockSpec((8, C), lambda i: (0, 0)),
            ],
            out_shape=[
                jax.ShapeDtypeStruct((N, C), _f32),
                jax.ShapeDtypeStruct((8, C), _f32),
            ],
        )(*ins)

    return run


_k3_c32 = _make_k3(32, True)
_k3_a16 = _make_k3(16, False)


def _k3_c64_body(aggA_ref, aggB_ref, q0_ref, q1_ref, q2_ref, q3_ref,
                 degT_ref, b_ref, h_ref, st_ref):
    i = pl.program_id(0)

    @pl.when(i == 0)
    def _():
        st_ref[...] = jnp.zeros_like(st_ref)

    deg = degT_ref[:, 0] + degT_ref[:, 1] + 1.0
    dinv = lax.rsqrt(deg)
    acc = jnp.concatenate([aggA_ref[0], aggA_ref[1],
                           aggB_ref[0], aggB_ref[1]], axis=-1)
    y = jnp.concatenate([q0_ref[...], q1_ref[...],
                         q2_ref[...], q3_ref[...]], axis=-1)
    h = dinv[:, None] * (acc + y) + b_ref[...]
    h_ref[...] = h
    st_ref[0:1, :] += jnp.sum(h, axis=0, keepdims=True)
    st_ref[1:2, :] += jnp.sum(h * h, axis=0, keepdims=True)


def _k3_c64(aggA, aggB, qs, degT, b):
    return pl.pallas_call(
        _k3_c64_body,
        grid=(GRID,),
        in_specs=[
            pl.BlockSpec((2, BLK, 16), lambda i: (0, i, 0)),
            pl.BlockSpec((2, BLK, 16), lambda i: (0, i, 0)),
            pl.BlockSpec((BLK, 16), lambda i: (i, 0)),
            pl.BlockSpec((BLK, 16), lambda i: (i, 0)),
            pl.BlockSpec((BLK, 16), lambda i: (i, 0)),
            pl.BlockSpec((BLK, 16), lambda i: (i, 0)),
            pl.BlockSpec((BLK, 2), lambda i: (i, 0)),
            pl.BlockSpec((1, 64), lambda i: (0, 0)),
        ],
        out_specs=[
            pl.BlockSpec((BLK, 64), lambda i: (i, 0)),
            pl.BlockSpec((8, 64), lambda i: (0, 0)),
        ],
        out_shape=[
            jax.ShapeDtypeStruct((N, 64), _f32),
            jax.ShapeDtypeStruct((8, 64), _f32),
        ],
    )(aggA, aggB, *qs, degT, b)


def _norm_x(h, st, gw, ga, gb):
    mean = st[0:1, :] * (1.0 / N)
    ex2 = st[1:2, :] * (1.0 / N)
    var = ex2 - ga * (2.0 - ga) * mean * mean
    xc = h - ga * mean
    xn = gw * xc * lax.rsqrt(var + 1e-5) + gb
    return jnp.maximum(xn, 0.0)


def _make_k4(C, Cn, n_out):
    Co = Cn // n_out

    def body(h_ref, st_ref, gw_ref, gb_ref, ga_ref, degT_ref, w_ref, *outs):
        deg = degT_ref[:, 0] + degT_ref[:, 1] + 1.0
        dinv = lax.rsqrt(deg)
        xn = _norm_x(h_ref[...], st_ref[...], gw_ref[...], ga_ref[...],
                     gb_ref[...])
        y = jnp.dot(xn, w_ref[...], preferred_element_type=_f32) * dinv[:, None]
        for t in range(n_out):
            outs[t][...] = y[:, t * Co:(t + 1) * Co]

    def run(h, st, gw, gb, ga, degT, w):
        return pl.pallas_call(
            body,
            grid=(GRID,),
            in_specs=[
                pl.BlockSpec((BLK, C), lambda i: (i, 0)),
                pl.BlockSpec((8, C), lambda i: (0, 0)),
                pl.BlockSpec((1, C), lambda i: (0, 0)),
                pl.BlockSpec((1, C), lambda i: (0, 0)),
                pl.BlockSpec((1, C), lambda i: (0, 0)),
                pl.BlockSpec((BLK, 2), lambda i: (i, 0)),
                pl.BlockSpec((C, Cn), lambda i: (0, 0)),
            ],
            out_specs=[pl.BlockSpec((BLK, Co), lambda i: (i, 0))] * n_out,
            out_shape=[jax.ShapeDtypeStruct((N, Co), _f32)] * n_out,
        )(h, st, gw, gb, ga, degT, w)

    return run


_k4_l0 = _make_k4(32, 16, 1)
_k4_l1 = _make_k4(16, 32, 2)
_k4_l2 = _make_k4(32, 64, 4)


def _k4_dec_body(h_ref, st_ref, gw_ref, gb_ref, ga_ref,
                 d0w_ref, d0b_ref, d1w_ref, d1b_ref, d2w_ref, d2b_ref,
                 d3w_ref, d3b_ref, out_ref):
    xn = _norm_x(h_ref[...], st_ref[...], gw_ref[...], ga_ref[...], gb_ref[...])
    d = jnp.maximum(jnp.dot(xn, d0w_ref[...], preferred_element_type=_f32)
                    + d0b_ref[...], 0.0)
    d = jnp.maximum(jnp.dot(d, d1w_ref[...], preferred_element_type=_f32)
                    + d1b_ref[...], 0.0)
    d = jnp.maximum(jnp.dot(d, d2w_ref[...], preferred_element_type=_f32)
                    + d2b_ref[...], 0.0)
    z = jnp.dot(d, d3w_ref[...], preferred_element_type=_f32) + d3b_ref[...]
    out_ref[...] = 1.0 / (1.0 + jnp.exp(-z))


def _k4_dec(h, st, gw, gb, ga, dws):
    return pl.pallas_call(
        _k4_dec_body,
        grid=(GRID,),
        in_specs=[
            pl.BlockSpec((BLK, HID), lambda i: (i, 0)),
            pl.BlockSpec((8, HID), lambda i: (0, 0)),
            pl.BlockSpec((1, HID), lambda i: (0, 0)),
            pl.BlockSpec((1, HID), lambda i: (0, 0)),
            pl.BlockSpec((1, HID), lambda i: (0, 0)),
            pl.BlockSpec((HID, 32), lambda i: (0, 0)),
            pl.BlockSpec((1, 32), lambda i: (0, 0)),
            pl.BlockSpec((32, 16), lambda i: (0, 0)),
            pl.BlockSpec((1, 16), lambda i: (0, 0)),
            pl.BlockSpec((16, 8), lambda i: (0, 0)),
            pl.BlockSpec((1, 8), lambda i: (0, 0)),
            pl.BlockSpec((8, 1), lambda i: (0, 0)),
            pl.BlockSpec((1, 1), lambda i: (0, 0)),
        ],
        out_specs=pl.BlockSpec((BLK, 1), lambda i: (i, 0)),
        out_shape=jax.ShapeDtypeStruct((N, 1), _f32),
    )(h, st, gw, gb, ga, *dws)


def kernel(local_x, voxel_x, label_hard, local_type, voxel_type, edge_index,
           params):
    p = params
    src = edge_index[0].astype(jnp.int32)
    dst = edge_index[1].astype(jnp.int32)
    npad = EPAD - E
    srcp = jnp.concatenate([src, jnp.zeros((npad,), jnp.int32)])
    dstp = jnp.concatenate([dst, jnp.full((npad,), N, jnp.int32)])
    src2d = srcp.reshape(EPAD // CHUNK, CHUNK)
    dst2d = dstp.reshape(EPAD // CHUNK, CHUNK)
    lt3 = local_type.astype(jnp.int32).reshape(LGRID, 1, LBLK)
    vt3 = voxel_type.astype(jnp.int32).reshape(GRID, 1, BLK)
    lb = label_hard[0]

    ones_c = jnp.ones((CHUNK,), _f32)
    zeros_1 = jnp.zeros((ZCH,), _f32)
    zeros_16 = jnp.zeros((ZCH, 16), _f32)

    deg0, deg1 = _deg_sc(dstp, ones_c, zeros_1)
    degT = jnp.stack([deg0, deg1], axis=1)

    t8 = _k1(lt3, local_x, p['mlp_W1'][:128])
    ylo0, yhi0 = _k2(vt3, voxel_x, lb, degT, t8,
                     p['mlp_W1'][128:256], p['mlp_W1'][256:],
                     p['mlp_b1'].reshape(1, HID),
                     p['mlp_W2'], p['mlp_b2'].reshape(1, HID),
                     p['conv0_W'])

    agg0 = _agg_cs16(ylo0, yhi0, src2d, dst2d, zeros_16)
    h0, st0 = _k3_c32(agg0, ylo0, yhi0, degT, p['conv0_b'].reshape(1, 32))
    y1 = _k4_l0(h0, st0, p['gn0_w'].reshape(1, 32), p['gn0_b'].reshape(1, 32),
                p['gn0_a'].reshape(1, 32), degT, p['conv1_W'])[0]

    agg1 = _agg_es16(y1, y1, src2d, dst2d, zeros_16)
    h1, st1 = _k3_a16(agg1, y1, None, degT, p['conv1_b'].reshape(1, 16))
    ylo2, yhi2 = _k4_l1(h1, st1, p['gn1_w'].reshape(1, 16),
                        p['gn1_b'].reshape(1, 16), p['gn1_a'].reshape(1, 16),
                        degT, p['conv2_W'])

    agg2 = _agg_cs16(ylo2, yhi2, src2d, dst2d, zeros_16)
    h2, st2 = _k3_c32(agg2, ylo2, yhi2, degT, p['conv2_b'].reshape(1, 32))
    qs = _k4_l2(h2, st2, p['gn2_w'].reshape(1, 32),
                p['gn2_b'].reshape(1, 32), p['gn2_a'].reshape(1, 32),
                degT, p['conv3_W'])

    agg3a = _agg_cs16(qs[0], qs[1], src2d, dst2d, zeros_16)
    agg3b = _agg_cs16(qs[2], qs[3], src2d, dst2d, zeros_16)
    h3, st3 = _k3_c64(agg3a, agg3b, qs, degT, p['conv3_b'].reshape(1, 64))
    out = _k4_dec(h3, st3, p['gn3_w'].reshape(1, HID),
                  p['gn3_b'].reshape(1, HID), p['gn3_a'].reshape(1, HID),
                  [p['dec0_W'], p['dec0_b'].reshape(1, 32),
                   p['dec1_W'], p['dec1_b'].reshape(1, 16),
                   p['dec2_W'], p['dec2_b'].reshape(1, 8),
                   p['dec3_W'], p['dec3_b'].reshape(1, 1)])
    return out


# speedup vs baseline: 21.3677x; 1.3436x over previous
import functools

import jax
import jax.numpy as jnp
from jax import lax
from jax.experimental import pallas as pl
from jax.experimental.pallas import tpu as pltpu
from jax.experimental.pallas import tpu_sc as plsc

N_LOCAL = 10000
N = 50000
E = 800000
NUM_TYPES = 8
HID = 64

NPAD = 51200
EPAD = 819200
CHUNK = 128
ZR = NPAD // 16
ZCH = 160
NZC = ZR // ZCH
SLAB = 40
HALF = 10

BLK = 2000
GRID = N // BLK
LBLK = 2000
LGRID = N_LOCAL // LBLK

RF = NPAD // 8
RFB = 256
RV = N // 8

_f32 = jnp.float32


def _mesh():
    return plsc.VectorSubcoreMesh(core_axis_name="c", subcore_axis_name="s",
                                  num_cores=2, num_subcores=16)


@functools.partial(
    pl.kernel,
    out_type=(jax.ShapeDtypeStruct((NPAD,), _f32),
              jax.ShapeDtypeStruct((NPAD,), _f32)),
    mesh=_mesh(),
    compiler_params=pltpu.CompilerParams(use_tc_tiling_on_sc=False),
    scratch_types=[
        pltpu.VMEM_SHARED((NPAD,), _f32),
        pltpu.VMEM((ZCH,), _f32),
        pltpu.VMEM((CHUNK,), _f32),
        pltpu.VMEM((2, CHUNK), jnp.int32),
        pltpu.SemaphoreType.DMA,
        pltpu.SemaphoreType.DMA,
        pltpu.SemaphoreType.DMA,
    ],
)
def _deg_sc(dst_hbm, ones_hbm, zeros_hbm, out0_hbm, out1_hbm,
            acc, zbuf, onesb, dstb, zsem, isem, ssem):
    c = lax.axis_index("c")
    s = lax.axis_index("s")
    pltpu.sync_copy(zeros_hbm, zbuf)
    zds = [pltpu.async_copy(zbuf, acc.at[pl.ds(s * ZR + k * ZCH, ZCH)], zsem)
           for k in range(NZC)]
    for d in zds:
        d.wait()
    plsc.subcore_barrier()
    pltpu.sync_copy(ones_hbm, onesb)
    wbase = (c * 16 + s) * (EPAD // 32)
    nch = EPAD // 32 // CHUNK

    def body(j, carry):
        par = j % 2
        base = wbase + j * CHUNK

        @pl.when(j >= 2)
        def _():
            pltpu.make_async_copy(zeros_hbm.at[pl.ds(0, CHUNK)],
                                  onesb, ssem).wait()

        pltpu.async_copy(dst_hbm.at[pl.ds(base, CHUNK)], dstb.at[par],
                         isem).wait()
        pltpu.async_copy(onesb, acc.at[dstb.at[par]], ssem, add=True)
        return carry

    lax.fori_loop(0, nch, body, 0)
    for _ in range(2):
        pltpu.make_async_copy(zeros_hbm.at[pl.ds(0, CHUNK)], onesb, ssem).wait()
    plsc.subcore_barrier()

    def copy_out(out_hbm):
        def cbody(k, carry):
            pltpu.sync_copy(acc.at[pl.ds(s * ZR + k * ZCH, ZCH)], zbuf)
            pltpu.sync_copy(zbuf, out_hbm.at[pl.ds(s * ZR + k * ZCH, ZCH)])
            return carry

        lax.fori_loop(0, NZC, cbody, 0)

    @pl.when(c == 0)
    def _():
        copy_out(out0_hbm)

    @pl.when(c == 1)
    def _():
        copy_out(out1_hbm)


def _make_agg(W, edge_split):
    sub_chunks = (EPAD // 32 if edge_split else EPAD // 16) // CHUNK
    nslab = sub_chunks // SLAB

    @functools.partial(
        pl.kernel,
        out_type=jax.ShapeDtypeStruct((2, NPAD, W), _f32),
        mesh=_mesh(),
        compiler_params=pltpu.CompilerParams(use_tc_tiling_on_sc=False),
        scratch_types=[
            pltpu.VMEM_SHARED((NPAD, W), _f32),
            pltpu.VMEM((ZCH, W), _f32),
            pltpu.VMEM((2, ZCH, W), _f32),
            pltpu.VMEM((SLAB, CHUNK), jnp.int32),
            pltpu.VMEM((2 * SLAB, CHUNK), jnp.int32),
            pltpu.VMEM((2 * HALF, CHUNK, W), _f32),
            pltpu.SemaphoreType.DMA,
            pltpu.SemaphoreType.DMA,
            pltpu.SemaphoreType.DMA,
            pltpu.SemaphoreType.DMA,
            pltpu.SemaphoreType.DMA,
            pltpu.SemaphoreType.DMA,
            pltpu.SemaphoreType.DMA,
            pltpu.SemaphoreType.DMA,
        ],
    )
    def k(ylo, yhi, src2d, dst2d, zeros_hbm, out_hbm,
          acc, zbuf, cbuf, src_slab, dst_slab, rows,
          gsA, gsB, ssA, ssB, zsem, rsem, ws0, ws1):
        c = lax.axis_index("c")
        s = lax.axis_index("s")
        gsems = (gsA, gsB)
        ssems = (ssA, ssB)
        wsems = (ws0, ws1)
        drain_src = ylo.at[pl.ds(0, CHUNK), :]

        pltpu.sync_copy(zeros_hbm, zbuf)
        zds = [pltpu.async_copy(zbuf, acc.at[pl.ds(s * ZR + k * ZCH, ZCH), :],
                                zsem)
               for k in range(NZC)]
        for d in zds:
            d.wait()
        plsc.subcore_barrier()

        if edge_split:
            base_row = (c * 16 + s) * sub_chunks
        else:
            base_row = s * sub_chunks

        def gather_half(tab, h, off):
            gds = [pltpu.async_copy(tab.at[src_slab.at[off + b]],
                                    rows.at[h * HALF + b], gsems[h])
                   for b in range(HALF)]
            for d in gds:
                d.wait()

        def slab_body(sl, carry):
            srow = base_row + sl * SLAB
            spar = (sl % 2) * SLAB
            pltpu.sync_copy(src2d.at[pl.ds(srow, SLAB), :], src_slab)
            pltpu.sync_copy(dst2d.at[pl.ds(srow, SLAB), :],
                            dst_slab.at[pl.ds(spar, SLAB), :])

            def grp_body(g, carry2):
                primed = (sl > 0) | (g > 0)
                for h in range(2):
                    off = g * (2 * HALF) + h * HALF
                    doff = spar + off

                    @pl.when(primed)
                    def _(h=h):
                        for b in range(HALF):
                            pltpu.make_async_copy(
                                drain_src, rows.at[h * HALF + b],
                                ssems[h]).wait()

                    if edge_split:
                        gather_half(ylo, h, off)
                    else:
                        @pl.when(c == 0)
                        def _(h=h, off=off):
                            gather_half(ylo, h, off)

                        @pl.when(c == 1)
                        def _(h=h, off=off):
                            gather_half(yhi, h, off)

                    for b in range(HALF):
                        pltpu.async_copy(rows.at[h * HALF + b],
                                         acc.at[dst_slab.at[doff + b]],
                                         ssems[h], add=True)
                return carry2

            lax.fori_loop(0, SLAB // (2 * HALF), grp_body, 0)
            return carry

        lax.fori_loop(0, nslab, slab_body, 0)
        for h in range(2):
            for b in range(HALF):
                pltpu.make_async_copy(drain_src, rows.at[h * HALF + b],
                                      ssems[h]).wait()
        plsc.subcore_barrier()

        wr = [None, None]
        for kk in range(NZC):
            par = kk % 2
            if wr[par] is not None:
                wr[par].wait()
            pltpu.async_copy(acc.at[pl.ds(s * ZR + kk * ZCH, ZCH), :],
                             cbuf.at[par], rsem).wait()
            wr[par] = pltpu.async_copy(
                cbuf.at[par], out_hbm.at[c, pl.ds(s * ZR + kk * ZCH, ZCH), :],
                wsems[par])
        wr[0].wait()
        wr[1].wait()

    return k


_agg_es16 = _make_agg(16, True)
_agg_cs16 = _make_agg(16, False)


def _k1_body(lt_ref, x_ref, w1a_ref, t8_ref, acc_ref, cnt_ref):
    i = pl.program_id(0)

    @pl.when(i == 0)
    def _():
        acc_ref[...] = jnp.zeros_like(acc_ref)
        cnt_ref[...] = jnp.zeros_like(cnt_ref)

    lt = lt_ref[0, 0, :]
    oh = (lt[:, None] == lax.broadcasted_iota(jnp.int32, (LBLK, NUM_TYPES), 1)
          ).astype(_f32)
    acc_ref[...] += lax.dot_general(oh, x_ref[...], (((0,), (0,)), ((), ())),
                                    preferred_element_type=_f32)
    cnt_ref[...] += jnp.sum(oh, axis=0, keepdims=True)

    @pl.when(i == pl.num_programs(0) - 1)
    def _():
        cnt = cnt_ref[0, :]
        tm = acc_ref[...] / jnp.maximum(cnt, 1.0)[:, None]
        tm = jnp.where(cnt[:, None] > 0, tm, 0.0)
        t8_ref[...] = jnp.dot(tm, w1a_ref[...], preferred_element_type=_f32)


def _k1(lt3, local_x, w1a):
    return pl.pallas_call(
        _k1_body,
        grid=(LGRID,),
        in_specs=[
            pl.BlockSpec((1, 1, LBLK), lambda i: (i, 0, 0)),
            pl.BlockSpec((LBLK, 128), lambda i: (i, 0)),
            pl.BlockSpec((128, HID), lambda i: (0, 0)),
        ],
        out_specs=pl.BlockSpec((NUM_TYPES, HID), lambda i: (0, 0)),
        out_shape=jax.ShapeDtypeStruct((NUM_TYPES, HID), _f32),
        scratch_shapes=[
            pltpu.VMEM((NUM_TYPES, 128), _f32),
            pltpu.VMEM((1, NUM_TYPES), _f32),
        ],
    )(lt3, local_x, w1a)


def _k2_body(vt_ref, vx_ref, lb_ref, degT_ref, t8_ref, w1b_ref, w1c_ref,
             b1_ref, w2_ref, b2_ref, wc0_ref, y_ref, y2_ref):
    deg = degT_ref[:, 0] + degT_ref[:, 1] + 1.0
    dinv = lax.rsqrt(deg)
    vt = vt_ref[0, 0, :]
    oh = (vt[:, None] == lax.broadcasted_iota(jnp.int32, (BLK, NUM_TYPES), 1)
          ).astype(_f32)
    m = jnp.dot(oh, t8_ref[...], preferred_element_type=_f32)
    h = m + jnp.dot(vx_ref[...], w1b_ref[...], preferred_element_type=_f32)
    h = h + jnp.dot(lb_ref[...], w1c_ref[...], preferred_element_type=_f32)
    h = jnp.maximum(h + b1_ref[...], 0.0)
    x = jnp.maximum(jnp.dot(h, w2_ref[...], preferred_element_type=_f32)
                    + b2_ref[...], 0.0)
    y = (jnp.dot(x, wc0_ref[...], preferred_element_type=_f32)
         * dinv[:, None])
    y_ref[...] = y[:, :16]
    y2_ref[...] = y[:, 16:]


def _k2(vt3, voxel_x, lb, degT, t8, w1b, w1c, b1, w2, b2, wc0):
    return pl.pallas_call(
        _k2_body,
        grid=(GRID,),
        in_specs=[
            pl.BlockSpec((1, 1, BLK), lambda i: (i, 0, 0)),
            pl.BlockSpec((BLK, 128), lambda i: (i, 0)),
            pl.BlockSpec((BLK, 16), lambda i: (i, 0)),
            pl.BlockSpec((BLK, 2), lambda i: (i, 0)),
            pl.BlockSpec((NUM_TYPES, HID), lambda i: (0, 0)),
            pl.BlockSpec((128, HID), lambda i: (0, 0)),
            pl.BlockSpec((16, HID), lambda i: (0, 0)),
            pl.BlockSpec((1, HID), lambda i: (0, 0)),
            pl.BlockSpec((HID, HID), lambda i: (0, 0)),
            pl.BlockSpec((1, HID), lambda i: (0, 0)),
            pl.BlockSpec((HID, 32), lambda i: (0, 0)),
        ],
        out_specs=[pl.BlockSpec((BLK, 16), lambda i: (i, 0))] * 2,
        out_shape=[jax.ShapeDtypeStruct((N, 16), _f32)] * 2,
    )(vt3, voxel_x, lb, degT, t8, w1b, w1c, b1, w2, b2, wc0)


def _make_k3h(pick):
    def body(agg_ref, y_ref, dinv_ref, b_ref, h_ref, st_ref):
        i = pl.program_id(0)

        @pl.when(i == 0)
        def _():
            st_ref[...] = jnp.zeros_like(st_ref)

        if pick is None:
            acc = agg_ref[0] + agg_ref[1]
        else:
            acc = agg_ref[0]
        h = dinv_ref[...] * (acc + y_ref[...]) + b_ref[...]
        h_ref[...] = h
        rowid = i * RFB + lax.broadcasted_iota(jnp.int32, (RFB, 128), 0)
        hm = jnp.where(rowid < RV, h, 0.0)
        st_ref[0:1, :] += jnp.sum(hm, axis=0, keepdims=True)
        st_ref[1:2, :] += jnp.sum(hm * hm, axis=0, keepdims=True)

    if pick is None:
        agg_spec = pl.BlockSpec((2, RFB, 128), lambda i: (0, i, 0))
    else:
        agg_spec = pl.BlockSpec((1, RFB, 128), lambda i, p=pick: (p, i, 0))

    def run(aggf, yf, dinvf, bf):
        return pl.pallas_call(
            body,
            grid=(RF // RFB,),
            in_specs=[
                agg_spec,
                pl.BlockSpec((RFB, 128), lambda i: (i, 0)),
                pl.BlockSpec((RFB, 128), lambda i: (i, 0)),
                pl.BlockSpec((1, 128), lambda i: (0, 0)),
            ],
            out_specs=[
                pl.BlockSpec((RFB, 128), lambda i: (i, 0)),
                pl.BlockSpec((8, 128), lambda i: (0, 0)),
            ],
            out_shape=[
                jax.ShapeDtypeStruct((RF, 128), _f32),
                jax.ShapeDtypeStruct((8, 128), _f32),
            ],
        )(aggf, yf, dinvf, bf)

    return run


_k3h_sum = _make_k3h(None)
_k3h_p0 = _make_k3h(0)
_k3h_p1 = _make_k3h(1)


def _kdinv_body(d0_ref, d1_ref, rt_ref, out_ref):
    deg8 = d0_ref[...] + d1_ref[...] + 1.0
    out_ref[...] = jnp.dot(lax.rsqrt(deg8), rt_ref[...],
                           preferred_element_type=_f32)


def _kdinv(d0f, d1f, rt16):
    return pl.pallas_call(
        _kdinv_body,
        grid=(RF // RFB,),
        in_specs=[
            pl.BlockSpec((RFB, 8), lambda i: (i, 0)),
            pl.BlockSpec((RFB, 8), lambda i: (i, 0)),
            pl.BlockSpec((8, 128), lambda i: (0, 0)),
        ],
        out_specs=pl.BlockSpec((RFB, 128), lambda i: (i, 0)),
        out_shape=jax.ShapeDtypeStruct((RF, 128), _f32),
    )(d0f, d1f, rt16)


def _xn_half(h, st, gw, gb, ga, F):
    mean = jnp.dot(st[0:1, :], F, preferred_element_type=_f32)
    ex2 = jnp.dot(st[1:2, :], F, preferred_element_type=_f32)
    var = ex2 - ga * (2.0 - ga) * mean * mean
    xc = h - ga * mean
    return jnp.maximum(gw * xc * lax.rsqrt(var + 1e-5) + gb, 0.0)


def _make_k4f(n_in, n_out):
    def body(*refs):
        hs = refs[:n_in]
        sts = refs[n_in:2 * n_in]
        gws = refs[2 * n_in:3 * n_in]
        gbs = refs[3 * n_in:4 * n_in]
        gas = refs[4 * n_in:5 * n_in]
        F_ref = refs[5 * n_in]
        dinv_ref = refs[5 * n_in + 1]
        wks = refs[5 * n_in + 2:5 * n_in + 2 + n_in * n_out]
        outs = refs[5 * n_in + 2 + n_in * n_out:]
        xs = [_xn_half(hs[i][...], sts[i][...], gws[i][...], gbs[i][...],
                       gas[i][...], F_ref[...]) for i in range(n_in)]
        for q in range(n_out):
            y = xs[0] @ wks[q][...]
            for i in range(1, n_in):
                y = y + xs[i] @ wks[i * n_out + q][...]
            outs[q][...] = y * dinv_ref[...]

    def run(hs, sts, gns, F, dinvf, wks):
        gws, gbs, gas = gns
        ins = (list(hs) + list(sts) + list(gws) + list(gbs) + list(gas)
               + [F, dinvf] + list(wks))
        return pl.pallas_call(
            body,
            grid=(RF // RFB,),
            in_specs=(
                [pl.BlockSpec((RFB, 128), lambda i: (i, 0))] * n_in
                + [pl.BlockSpec((8, 128), lambda i: (0, 0))] * n_in
                + [pl.BlockSpec((1, 128), lambda i: (0, 0))] * (3 * n_in)
                + [pl.BlockSpec((128, 128), lambda i: (0, 0))]
                + [pl.BlockSpec((RFB, 128), lambda i: (i, 0))]
                + [pl.BlockSpec((128, 128), lambda i: (0, 0))] * (n_in * n_out)
            ),
            out_specs=[pl.BlockSpec((RFB, 128), lambda i: (i, 0))] * n_out,
            out_shape=[jax.ShapeDtypeStruct((RF, 128), _f32)] * n_out,
        )(*ins)

    return run


_k4f_21 = _make_k4f(2, 1)
_k4f_12 = _make_k4f(1, 2)
_k4f_24 = _make_k4f(2, 4)


def _k4dec_body(*refs):
    hs = refs[:4]
    sts = refs[4:8]
    gws = refs[8:12]
    gbs = refs[12:16]
    gas = refs[16:20]
    F_ref = refs[20]
    d0k = refs[21:25]
    b0f, d1k, b1f, d2k, b2f, d3k, b3f = refs[25:32]
    out_ref = refs[32]
    xs = [_xn_half(hs[i][...], sts[i][...], gws[i][...], gbs[i][...],
                   gas[i][...], F_ref[...]) for i in range(4)]
    d = xs[0] @ d0k[0][...]
    for i in range(1, 4):
        d = d + xs[i] @ d0k[i][...]
    d = jnp.maximum(d + b0f[...], 0.0)
    d = jnp.maximum(d @ d1k[...] + b1f[...], 0.0)
    d = jnp.maximum(d @ d2k[...] + b2f[...], 0.0)
    z = d @ d3k[...] + b3f[...]
    out_ref[...] = 1.0 / (1.0 + jnp.exp(-z))


def _k4dec(hs, sts, gns, F, dws):
    gws, gbs, gas = gns
    d0k0, d0k1, d0k2, d0k3, b0f, d1k, b1f, d2k, b2f, d3k, b3f = dws
    ins = (list(hs) + list(sts) + list(gws) + list(gbs) + list(gas)
           + [F, d0k0, d0k1, d0k2, d0k3, b0f, d1k, b1f, d2k, b2f, d3k, b3f])
    return pl.pallas_call(
        _k4dec_body,
        grid=(RF // RFB,),
        in_specs=(
            [pl.BlockSpec((RFB, 128), lambda i: (i, 0))] * 4
            + [pl.BlockSpec((8, 128), lambda i: (0, 0))] * 4
            + [pl.BlockSpec((1, 128), lambda i: (0, 0))] * 12
            + [pl.BlockSpec((128, 128), lambda i: (0, 0))]
            + [pl.BlockSpec((128, 256), lambda i: (0, 0))] * 4
            + [pl.BlockSpec((1, 256), lambda i: (0, 0))]
            + [pl.BlockSpec((256, 128), lambda i: (0, 0))]
            + [pl.BlockSpec((1, 128), lambda i: (0, 0))]
            + [pl.BlockSpec((128, 64), lambda i: (0, 0))]
            + [pl.BlockSpec((1, 64), lambda i: (0, 0))]
            + [pl.BlockSpec((64, 8), lambda i: (0, 0))]
            + [pl.BlockSpec((1, 8), lambda i: (0, 0))]
        ),
        out_specs=pl.BlockSpec((RFB, 8), lambda i: (i, 0)),
        out_shape=jax.ShapeDtypeStruct((RF, 8), _f32),
    )(*ins)


def kernel(local_x, voxel_x, label_hard, local_type, voxel_type, edge_index,
           params):
    p = params
    src = edge_index[0].astype(jnp.int32)
    dst = edge_index[1].astype(jnp.int32)
    npad = EPAD - E
    srcp = jnp.concatenate([src, jnp.zeros((npad,), jnp.int32)])
    dstp = jnp.concatenate([dst, jnp.full((npad,), N, jnp.int32)])
    src2d = srcp.reshape(EPAD // CHUNK, CHUNK)
    dst2d = dstp.reshape(EPAD // CHUNK, CHUNK)
    lt3 = local_type.astype(jnp.int32).reshape(LGRID, 1, LBLK)
    vt3 = voxel_type.astype(jnp.int32).reshape(GRID, 1, BLK)
    lb = label_hard[0]

    ones_c = jnp.ones((CHUNK,), _f32)
    zeros_1 = jnp.zeros((ZCH,), _f32)
    zeros_16 = jnp.zeros((ZCH, 16), _f32)

    eye8 = jnp.eye(8, dtype=_f32)

    def k8(w):
        return jnp.kron(eye8, w)

    def tile8(v):
        return jnp.tile(v, 8).reshape(1, -1)

    def halves(v, n):
        return [v[16 * i:16 * (i + 1)] for i in range(n)]

    lanes = jnp.arange(128)
    rt16 = (lanes[None, :] // 16 == jnp.arange(8)[:, None]).astype(_f32)
    F = (lanes[:, None] % 16 == lanes[None, :] % 16).astype(_f32) / N

    deg0, deg1 = _deg_sc(dstp, ones_c, zeros_1)
    degT = jnp.stack([deg0, deg1], axis=1)
    dinvf = _kdinv(deg0.reshape(RF, 8), deg1.reshape(RF, 8), rt16)

    t8 = _k1(lt3, local_x, p['mlp_W1'][:128])
    ylo0, yhi0 = _k2(vt3, voxel_x, lb, degT, t8,
                     p['mlp_W1'][128:256], p['mlp_W1'][256:],
                     p['mlp_b1'].reshape(1, HID),
                     p['mlp_W2'], p['mlp_b2'].reshape(1, HID),
                     p['conv0_W'])

    def gnf(li, n):
        return ([tile8(v) for v in halves(p['gn%d_w' % li], n)],
                [tile8(v) for v in halves(p['gn%d_b' % li], n)],
                [tile8(v) for v in halves(p['gn%d_a' % li], n)])

    def wkron(w, n_in, n_out):
        return [k8(w[16 * i:16 * (i + 1), 16 * q:16 * (q + 1)])
                for i in range(n_in) for q in range(n_out)]

    agg0 = _agg_cs16(ylo0, yhi0, src2d, dst2d, zeros_16)
    agg0f = agg0.reshape(2, RF, 128)
    b0 = halves(p['conv0_b'], 2)
    h00, st00 = _k3h_p0(agg0f, ylo0.reshape(RV, 128), dinvf, tile8(b0[0]))
    h01, st01 = _k3h_p1(agg0f, yhi0.reshape(RV, 128), dinvf, tile8(b0[1]))
    y1f = _k4f_21([h00, h01], [st00, st01], gnf(0, 2), F, dinvf,
                  wkron(p['conv1_W'], 2, 1))[0]

    agg1 = _agg_es16(y1f.reshape(NPAD, 16), y1f.reshape(NPAD, 16),
                     src2d, dst2d, zeros_16)
    h1, st1 = _k3h_sum(agg1.reshape(2, RF, 128), y1f, dinvf,
                       tile8(p['conv1_b']))
    y2lof, y2hif = _k4f_12([h1], [st1], gnf(1, 1), F, dinvf,
                           wkron(p['conv2_W'], 1, 2))

    agg2 = _agg_cs16(y2lof.reshape(NPAD, 16), y2hif.reshape(NPAD, 16),
                     src2d, dst2d, zeros_16)
    agg2f = agg2.reshape(2, RF, 128)
    b2 = halves(p['conv2_b'], 2)
    h20, st20 = _k3h_p0(agg2f, y2lof, dinvf, tile8(b2[0]))
    h21, st21 = _k3h_p1(agg2f, y2hif, dinvf, tile8(b2[1]))
    qs = _k4f_24([h20, h21], [st20, st21], gnf(2, 2), F, dinvf,
                 wkron(p['conv3_W'], 2, 4))

    agg3a = _agg_cs16(qs[0].reshape(NPAD, 16), qs[1].reshape(NPAD, 16),
                      src2d, dst2d, zeros_16)
    agg3b = _agg_cs16(qs[2].reshape(NPAD, 16), qs[3].reshape(NPAD, 16),
                      src2d, dst2d, zeros_16)
    agg3af = agg3a.reshape(2, RF, 128)
    agg3bf = agg3b.reshape(2, RF, 128)
    b3 = halves(p['conv3_b'], 4)
    h30, st30 = _k3h_p0(agg3af, qs[0], dinvf, tile8(b3[0]))
    h31, st31 = _k3h_p1(agg3af, qs[1], dinvf, tile8(b3[1]))
    h32, st32 = _k3h_p0(agg3bf, qs[2], dinvf, tile8(b3[2]))
    h33, st33 = _k3h_p1(agg3bf, qs[3], dinvf, tile8(b3[3]))
    dws = ([k8(p['dec0_W'][16 * i:16 * (i + 1), :]) for i in range(4)]
           + [tile8(p['dec0_b']), k8(p['dec1_W']), tile8(p['dec1_b']),
              k8(p['dec2_W']), tile8(p['dec2_b']), k8(p['dec3_W']),
              tile8(p['dec3_b'])])
    outf = _k4dec([h30, h31, h32, h33], [st30, st31, st32, st33],
                  gnf(3, 4), F, dws)
    return outf.reshape(NPAD, 1)[:N]


# speedup vs baseline: 23.0025x; 1.0765x over previous
import functools

import jax
import jax.numpy as jnp
from jax import lax
from jax.experimental import pallas as pl
from jax.experimental.pallas import tpu as pltpu
from jax.experimental.pallas import tpu_sc as plsc

N_LOCAL = 10000
N = 50000
E = 800000
NUM_TYPES = 8
HID = 64

NPAD = 51200
EPAD = 819200
CHUNK = 128
ZR = NPAD // 16
ZCH = 160
NZC = ZR // ZCH
SLAB = 40
HALF = 10

BLK = 2000
GRID = N // BLK
LBLK = 2000
LGRID = N_LOCAL // LBLK

RF = NPAD // 8
RFB = 256
RV = N // 8

_f32 = jnp.float32


def _mesh():
    return plsc.VectorSubcoreMesh(core_axis_name="c", subcore_axis_name="s",
                                  num_cores=2, num_subcores=16)


@functools.partial(
    pl.kernel,
    out_type=(jax.ShapeDtypeStruct((NPAD,), _f32),
              jax.ShapeDtypeStruct((NPAD,), _f32)),
    mesh=_mesh(),
    compiler_params=pltpu.CompilerParams(use_tc_tiling_on_sc=False),
    scratch_types=[
        pltpu.VMEM_SHARED((NPAD,), _f32),
        pltpu.VMEM((ZCH,), _f32),
        pltpu.VMEM((CHUNK,), _f32),
        pltpu.VMEM((2, CHUNK), jnp.int32),
        pltpu.SemaphoreType.DMA,
        pltpu.SemaphoreType.DMA,
        pltpu.SemaphoreType.DMA,
    ],
)
def _deg_sc(dst_hbm, ones_hbm, zeros_hbm, out0_hbm, out1_hbm,
            acc, zbuf, onesb, dstb, zsem, isem, ssem):
    c = lax.axis_index("c")
    s = lax.axis_index("s")
    pltpu.sync_copy(zeros_hbm, zbuf)
    zds = [pltpu.async_copy(zbuf, acc.at[pl.ds(s * ZR + k * ZCH, ZCH)], zsem)
           for k in range(NZC)]
    for d in zds:
        d.wait()
    plsc.subcore_barrier()
    pltpu.sync_copy(ones_hbm, onesb)
    wbase = (c * 16 + s) * (EPAD // 32)
    nch = EPAD // 32 // CHUNK

    def body(j, carry):
        par = j % 2
        base = wbase + j * CHUNK

        @pl.when(j >= 2)
        def _():
            pltpu.make_async_copy(zeros_hbm.at[pl.ds(0, CHUNK)],
                                  onesb, ssem).wait()

        pltpu.async_copy(dst_hbm.at[pl.ds(base, CHUNK)], dstb.at[par],
                         isem).wait()
        pltpu.async_copy(onesb, acc.at[dstb.at[par]], ssem, add=True)
        return carry

    lax.fori_loop(0, nch, body, 0)
    for _ in range(2):
        pltpu.make_async_copy(zeros_hbm.at[pl.ds(0, CHUNK)], onesb, ssem).wait()
    plsc.subcore_barrier()

    def copy_out(out_hbm):
        def cbody(k, carry):
            pltpu.sync_copy(acc.at[pl.ds(s * ZR + k * ZCH, ZCH)], zbuf)
            pltpu.sync_copy(zbuf, out_hbm.at[pl.ds(s * ZR + k * ZCH, ZCH)])
            return carry

        lax.fori_loop(0, NZC, cbody, 0)

    @pl.when(c == 0)
    def _():
        copy_out(out0_hbm)

    @pl.when(c == 1)
    def _():
        copy_out(out1_hbm)


def _make_agg(W, edge_split):
    sub_chunks = (EPAD // 32 if edge_split else EPAD // 16) // CHUNK
    nslab = sub_chunks // SLAB

    @functools.partial(
        pl.kernel,
        out_type=jax.ShapeDtypeStruct((2, NPAD, W), _f32),
        mesh=_mesh(),
        compiler_params=pltpu.CompilerParams(use_tc_tiling_on_sc=False),
        scratch_types=[
            pltpu.VMEM_SHARED((NPAD, W), _f32),
            pltpu.VMEM((ZCH, W), _f32),
            pltpu.VMEM((2, ZCH, W), _f32),
            pltpu.VMEM((SLAB, CHUNK), jnp.int32),
            pltpu.VMEM((2 * SLAB, CHUNK), jnp.int32),
            pltpu.VMEM((2 * HALF, CHUNK, W), _f32),
            pltpu.SemaphoreType.DMA,
            pltpu.SemaphoreType.DMA,
            pltpu.SemaphoreType.DMA,
            pltpu.SemaphoreType.DMA,
            pltpu.SemaphoreType.DMA,
            pltpu.SemaphoreType.DMA,
            pltpu.SemaphoreType.DMA,
            pltpu.SemaphoreType.DMA,
        ],
    )
    def k(ylo, yhi, src2d, dst2d, zeros_hbm, out_hbm,
          acc, zbuf, cbuf, src_slab, dst_slab, rows,
          gsA, gsB, ssA, ssB, zsem, rsem, ws0, ws1):
        c = lax.axis_index("c")
        s = lax.axis_index("s")
        gsems = (gsA, gsB)
        ssems = (ssA, ssB)
        wsems = (ws0, ws1)
        drain_src = ylo.at[pl.ds(0, CHUNK), :]

        pltpu.sync_copy(zeros_hbm, zbuf)
        zds = [pltpu.async_copy(zbuf, acc.at[pl.ds(s * ZR + k * ZCH, ZCH), :],
                                zsem)
               for k in range(NZC)]
        for d in zds:
            d.wait()
        plsc.subcore_barrier()

        if edge_split:
            base_row = (c * 16 + s) * sub_chunks
        else:
            base_row = s * sub_chunks

        def gather_half(tab, h, off):
            gds = [pltpu.async_copy(tab.at[src_slab.at[off + b]],
                                    rows.at[h * HALF + b], gsems[h])
                   for b in range(HALF)]
            for d in gds:
                d.wait()

        def slab_body(sl, carry):
            srow = base_row + sl * SLAB
            spar = (sl % 2) * SLAB
            pltpu.sync_copy(src2d.at[pl.ds(srow, SLAB), :], src_slab)
            pltpu.sync_copy(dst2d.at[pl.ds(srow, SLAB), :],
                            dst_slab.at[pl.ds(spar, SLAB), :])

            def grp_body(g, carry2):
                primed = (sl > 0) | (g > 0)
                for h in range(2):
                    off = g * (2 * HALF) + h * HALF
                    doff = spar + off

                    @pl.when(primed)
                    def _(h=h):
                        for b in range(HALF):
                            pltpu.make_async_copy(
                                drain_src, rows.at[h * HALF + b],
                                ssems[h]).wait()

                    if edge_split:
                        gather_half(ylo, h, off)
                    else:
                        @pl.when(c == 0)
                        def _(h=h, off=off):
                            gather_half(ylo, h, off)

                        @pl.when(c == 1)
                        def _(h=h, off=off):
                            gather_half(yhi, h, off)

                    for b in range(HALF):
                        pltpu.async_copy(rows.at[h * HALF + b],
                                         acc.at[dst_slab.at[doff + b]],
                                         ssems[h], add=True)
                return carry2

            lax.fori_loop(0, SLAB // (2 * HALF), grp_body, 0)
            return carry

        lax.fori_loop(0, nslab, slab_body, 0)
        for h in range(2):
            for b in range(HALF):
                pltpu.make_async_copy(drain_src, rows.at[h * HALF + b],
                                      ssems[h]).wait()
        plsc.subcore_barrier()

        rsems = (rsem, zsem)
        rd = [None, None]
        wr = [None, None]
        rd[0] = pltpu.async_copy(acc.at[pl.ds(s * ZR, ZCH), :],
                                 cbuf.at[0], rsems[0])
        for kk in range(NZC):
            par = kk % 2
            nb = (kk + 1) % 2
            if kk + 1 < NZC:
                if wr[nb] is not None:
                    wr[nb].wait()
                    wr[nb] = None
                rd[nb] = pltpu.async_copy(
                    acc.at[pl.ds(s * ZR + (kk + 1) * ZCH, ZCH), :],
                    cbuf.at[nb], rsems[nb])
            rd[par].wait()
            wr[par] = pltpu.async_copy(
                cbuf.at[par], out_hbm.at[c, pl.ds(s * ZR + kk * ZCH, ZCH), :],
                wsems[par])
        wr[0].wait()
        wr[1].wait()

    return k


_agg_es16 = _make_agg(16, True)
_agg_cs16 = _make_agg(16, False)


def _k1_body(lt_ref, x_ref, w1a_ref, t8_ref, acc_ref, cnt_ref):
    i = pl.program_id(0)

    @pl.when(i == 0)
    def _():
        acc_ref[...] = jnp.zeros_like(acc_ref)
        cnt_ref[...] = jnp.zeros_like(cnt_ref)

    lt = lt_ref[0, 0, :]
    oh = (lt[:, None] == lax.broadcasted_iota(jnp.int32, (LBLK, NUM_TYPES), 1)
          ).astype(_f32)
    acc_ref[...] += lax.dot_general(oh, x_ref[...], (((0,), (0,)), ((), ())),
                                    preferred_element_type=_f32)
    cnt_ref[...] += jnp.sum(oh, axis=0, keepdims=True)

    @pl.when(i == pl.num_programs(0) - 1)
    def _():
        cnt = cnt_ref[0, :]
        tm = acc_ref[...] / jnp.maximum(cnt, 1.0)[:, None]
        tm = jnp.where(cnt[:, None] > 0, tm, 0.0)
        t8_ref[...] = jnp.dot(tm, w1a_ref[...], preferred_element_type=_f32)


def _k1(lt3, local_x, w1a):
    return pl.pallas_call(
        _k1_body,
        grid=(LGRID,),
        in_specs=[
            pl.BlockSpec((1, 1, LBLK), lambda i: (i, 0, 0)),
            pl.BlockSpec((LBLK, 128), lambda i: (i, 0)),
            pl.BlockSpec((128, HID), lambda i: (0, 0)),
        ],
        out_specs=pl.BlockSpec((NUM_TYPES, HID), lambda i: (0, 0)),
        out_shape=jax.ShapeDtypeStruct((NUM_TYPES, HID), _f32),
        scratch_shapes=[
            pltpu.VMEM((NUM_TYPES, 128), _f32),
            pltpu.VMEM((1, NUM_TYPES), _f32),
        ],
    )(lt3, local_x, w1a)


def _k2_body(vt_ref, vx_ref, lb_ref, degT_ref, t8_ref, w1b_ref, w1c_ref,
             b1_ref, w2_ref, b2_ref, wc0_ref, y_ref, y2_ref):
    deg = degT_ref[:, 0] + degT_ref[:, 1] + 1.0
    dinv = lax.rsqrt(deg)
    vt = vt_ref[0, 0, :]
    oh = (vt[:, None] == lax.broadcasted_iota(jnp.int32, (BLK, NUM_TYPES), 1)
          ).astype(_f32)
    m = jnp.dot(oh, t8_ref[...], preferred_element_type=_f32)
    h = m + jnp.dot(vx_ref[...], w1b_ref[...], preferred_element_type=_f32)
    h = h + jnp.dot(lb_ref[...], w1c_ref[...], preferred_element_type=_f32)
    h = jnp.maximum(h + b1_ref[...], 0.0)
    x = jnp.maximum(jnp.dot(h, w2_ref[...], preferred_element_type=_f32)
                    + b2_ref[...], 0.0)
    y = (jnp.dot(x, wc0_ref[...], preferred_element_type=_f32)
         * dinv[:, None])
    y_ref[...] = y[:, :16]
    y2_ref[...] = y[:, 16:]


def _k2(vt3, voxel_x, lb, degT, t8, w1b, w1c, b1, w2, b2, wc0):
    return pl.pallas_call(
        _k2_body,
        grid=(GRID,),
        in_specs=[
            pl.BlockSpec((1, 1, BLK), lambda i: (i, 0, 0)),
            pl.BlockSpec((BLK, 128), lambda i: (i, 0)),
            pl.BlockSpec((BLK, 16), lambda i: (i, 0)),
            pl.BlockSpec((BLK, 2), lambda i: (i, 0)),
            pl.BlockSpec((NUM_TYPES, HID), lambda i: (0, 0)),
            pl.BlockSpec((128, HID), lambda i: (0, 0)),
            pl.BlockSpec((16, HID), lambda i: (0, 0)),
            pl.BlockSpec((1, HID), lambda i: (0, 0)),
            pl.BlockSpec((HID, HID), lambda i: (0, 0)),
            pl.BlockSpec((1, HID), lambda i: (0, 0)),
            pl.BlockSpec((HID, 32), lambda i: (0, 0)),
        ],
        out_specs=[pl.BlockSpec((BLK, 16), lambda i: (i, 0))] * 2,
        out_shape=[jax.ShapeDtypeStruct((N, 16), _f32)] * 2,
    )(vt3, voxel_x, lb, degT, t8, w1b, w1c, b1, w2, b2, wc0)


def _make_k3h(pick):
    def body(agg_ref, y_ref, dinv_ref, b_ref, h_ref, st_ref):
        i = pl.program_id(0)

        @pl.when(i == 0)
        def _():
            st_ref[...] = jnp.zeros_like(st_ref)

        if pick is None:
            acc = agg_ref[0] + agg_ref[1]
        else:
            acc = agg_ref[0]
        h = dinv_ref[...] * (acc + y_ref[...]) + b_ref[...]
        h_ref[...] = h
        rowid = i * RFB + lax.broadcasted_iota(jnp.int32, (RFB, 128), 0)
        hm = jnp.where(rowid < RV, h, 0.0)
        st_ref[0:1, :] += jnp.sum(hm, axis=0, keepdims=True)
        st_ref[1:2, :] += jnp.sum(hm * hm, axis=0, keepdims=True)

    if pick is None:
        agg_spec = pl.BlockSpec((2, RFB, 128), lambda i: (0, i, 0))
    else:
        agg_spec = pl.BlockSpec((1, RFB, 128), lambda i, p=pick: (p, i, 0))

    def run(aggf, yf, dinvf, bf):
        return pl.pallas_call(
            body,
            grid=(RF // RFB,),
            in_specs=[
                agg_spec,
                pl.BlockSpec((RFB, 128), lambda i: (i, 0)),
                pl.BlockSpec((RFB, 128), lambda i: (i, 0)),
                pl.BlockSpec((1, 128), lambda i: (0, 0)),
            ],
            out_specs=[
                pl.BlockSpec((RFB, 128), lambda i: (i, 0)),
                pl.BlockSpec((8, 128), lambda i: (0, 0)),
            ],
            out_shape=[
                jax.ShapeDtypeStruct((RF, 128), _f32),
                jax.ShapeDtypeStruct((8, 128), _f32),
            ],
        )(aggf, yf, dinvf, bf)

    return run


_k3h_sum = _make_k3h(None)


def _k3h2_body(agg_ref, y0_ref, y1_ref, dinv_ref, b0_ref, b1_ref,
               h0_ref, h1_ref, st0_ref, st1_ref):
    i = pl.program_id(0)

    @pl.when(i == 0)
    def _():
        st0_ref[...] = jnp.zeros_like(st0_ref)
        st1_ref[...] = jnp.zeros_like(st1_ref)

    dinv = dinv_ref[...]
    rowid = i * RFB + lax.broadcasted_iota(jnp.int32, (RFB, 128), 0)
    valid = rowid < RV
    for agg, y_ref, b_ref, h_ref, st_ref in (
            (agg_ref[0], y0_ref, b0_ref, h0_ref, st0_ref),
            (agg_ref[1], y1_ref, b1_ref, h1_ref, st1_ref)):
        h = dinv * (agg + y_ref[...]) + b_ref[...]
        h_ref[...] = h
        hm = jnp.where(valid, h, 0.0)
        st_ref[0:1, :] += jnp.sum(hm, axis=0, keepdims=True)
        st_ref[1:2, :] += jnp.sum(hm * hm, axis=0, keepdims=True)


def _k3h2(aggf, y0f, y1f, dinvf, b0f, b1f):
    return pl.pallas_call(
        _k3h2_body,
        grid=(RF // RFB,),
        in_specs=[
            pl.BlockSpec((2, RFB, 128), lambda i: (0, i, 0)),
            pl.BlockSpec((RFB, 128), lambda i: (i, 0)),
            pl.BlockSpec((RFB, 128), lambda i: (i, 0)),
            pl.BlockSpec((RFB, 128), lambda i: (i, 0)),
            pl.BlockSpec((1, 128), lambda i: (0, 0)),
            pl.BlockSpec((1, 128), lambda i: (0, 0)),
        ],
        out_specs=[
            pl.BlockSpec((RFB, 128), lambda i: (i, 0)),
            pl.BlockSpec((RFB, 128), lambda i: (i, 0)),
            pl.BlockSpec((8, 128), lambda i: (0, 0)),
            pl.BlockSpec((8, 128), lambda i: (0, 0)),
        ],
        out_shape=[
            jax.ShapeDtypeStruct((RF, 128), _f32),
            jax.ShapeDtypeStruct((RF, 128), _f32),
            jax.ShapeDtypeStruct((8, 128), _f32),
            jax.ShapeDtypeStruct((8, 128), _f32),
        ],
    )(aggf, y0f, y1f, dinvf, b0f, b1f)


def _kdinv_body(d0_ref, d1_ref, rt_ref, out_ref):
    deg8 = d0_ref[...] + d1_ref[...] + 1.0
    out_ref[...] = jnp.dot(lax.rsqrt(deg8), rt_ref[...],
                           preferred_element_type=_f32)


def _kdinv(d0f, d1f, rt16):
    return pl.pallas_call(
        _kdinv_body,
        grid=(RF // RFB,),
        in_specs=[
            pl.BlockSpec((RFB, 8), lambda i: (i, 0)),
            pl.BlockSpec((RFB, 8), lambda i: (i, 0)),
            pl.BlockSpec((8, 128), lambda i: (0, 0)),
        ],
        out_specs=pl.BlockSpec((RFB, 128), lambda i: (i, 0)),
        out_shape=jax.ShapeDtypeStruct((RF, 128), _f32),
    )(d0f, d1f, rt16)


def _xn_half(h, st, gw, gb, ga, F):
    mean = jnp.dot(st[0:1, :], F, preferred_element_type=_f32)
    ex2 = jnp.dot(st[1:2, :], F, preferred_element_type=_f32)
    var = ex2 - ga * (2.0 - ga) * mean * mean
    xc = h - ga * mean
    return jnp.maximum(gw * xc * lax.rsqrt(var + 1e-5) + gb, 0.0)


def _make_k4f(n_in, n_out):
    def body(*refs):
        hs = refs[:n_in]
        sts = refs[n_in:2 * n_in]
        gws = refs[2 * n_in:3 * n_in]
        gbs = refs[3 * n_in:4 * n_in]
        gas = refs[4 * n_in:5 * n_in]
        F_ref = refs[5 * n_in]
        dinv_ref = refs[5 * n_in + 1]
        wks = refs[5 * n_in + 2:5 * n_in + 2 + n_in * n_out]
        outs = refs[5 * n_in + 2 + n_in * n_out:]
        xs = [_xn_half(hs[i][...], sts[i][...], gws[i][...], gbs[i][...],
                       gas[i][...], F_ref[...]) for i in range(n_in)]
        for q in range(n_out):
            y = xs[0] @ wks[q][...]
            for i in range(1, n_in):
                y = y + xs[i] @ wks[i * n_out + q][...]
            outs[q][...] = y * dinv_ref[...]

    def run(hs, sts, gns, F, dinvf, wks):
        gws, gbs, gas = gns
        ins = (list(hs) + list(sts) + list(gws) + list(gbs) + list(gas)
               + [F, dinvf] + list(wks))
        return pl.pallas_call(
            body,
            grid=(RF // RFB,),
            in_specs=(
                [pl.BlockSpec((RFB, 128), lambda i: (i, 0))] * n_in
                + [pl.BlockSpec((8, 128), lambda i: (0, 0))] * n_in
                + [pl.BlockSpec((1, 128), lambda i: (0, 0))] * (3 * n_in)
                + [pl.BlockSpec((128, 128), lambda i: (0, 0))]
                + [pl.BlockSpec((RFB, 128), lambda i: (i, 0))]
                + [pl.BlockSpec((128, 128), lambda i: (0, 0))] * (n_in * n_out)
            ),
            out_specs=[pl.BlockSpec((RFB, 128), lambda i: (i, 0))] * n_out,
            out_shape=[jax.ShapeDtypeStruct((RF, 128), _f32)] * n_out,
        )(*ins)

    return run


_k4f_21 = _make_k4f(2, 1)
_k4f_12 = _make_k4f(1, 2)
_k4f_24 = _make_k4f(2, 4)


def _k4dec_body(*refs):
    hs = refs[:4]
    sts = refs[4:8]
    gws = refs[8:12]
    gbs = refs[12:16]
    gas = refs[16:20]
    F_ref = refs[20]
    d0k = refs[21:25]
    b0f, d1k, b1f, d2k, b2f, d3k, b3f = refs[25:32]
    out_ref = refs[32]
    xs = [_xn_half(hs[i][...], sts[i][...], gws[i][...], gbs[i][...],
                   gas[i][...], F_ref[...]) for i in range(4)]
    d = xs[0] @ d0k[0][...]
    for i in range(1, 4):
        d = d + xs[i] @ d0k[i][...]
    d = jnp.maximum(d + b0f[...], 0.0)
    d = jnp.maximum(d @ d1k[...] + b1f[...], 0.0)
    d = jnp.maximum(d @ d2k[...] + b2f[...], 0.0)
    z = d @ d3k[...] + b3f[...]
    out_ref[...] = 1.0 / (1.0 + jnp.exp(-z))


def _k4dec(hs, sts, gns, F, dws):
    gws, gbs, gas = gns
    d0k0, d0k1, d0k2, d0k3, b0f, d1k, b1f, d2k, b2f, d3k, b3f = dws
    ins = (list(hs) + list(sts) + list(gws) + list(gbs) + list(gas)
           + [F, d0k0, d0k1, d0k2, d0k3, b0f, d1k, b1f, d2k, b2f, d3k, b3f])
    return pl.pallas_call(
        _k4dec_body,
        grid=(RF // RFB,),
        in_specs=(
            [pl.BlockSpec((RFB, 128), lambda i: (i, 0))] * 4
            + [pl.BlockSpec((8, 128), lambda i: (0, 0))] * 4
            + [pl.BlockSpec((1, 128), lambda i: (0, 0))] * 12
            + [pl.BlockSpec((128, 128), lambda i: (0, 0))]
            + [pl.BlockSpec((128, 256), lambda i: (0, 0))] * 4
            + [pl.BlockSpec((1, 256), lambda i: (0, 0))]
            + [pl.BlockSpec((256, 128), lambda i: (0, 0))]
            + [pl.BlockSpec((1, 128), lambda i: (0, 0))]
            + [pl.BlockSpec((128, 64), lambda i: (0, 0))]
            + [pl.BlockSpec((1, 64), lambda i: (0, 0))]
            + [pl.BlockSpec((64, 8), lambda i: (0, 0))]
            + [pl.BlockSpec((1, 8), lambda i: (0, 0))]
        ),
        out_specs=pl.BlockSpec((RFB, 8), lambda i: (i, 0)),
        out_shape=jax.ShapeDtypeStruct((RF, 8), _f32),
    )(*ins)


def kernel(local_x, voxel_x, label_hard, local_type, voxel_type, edge_index,
           params):
    p = params
    src = edge_index[0].astype(jnp.int32)
    dst = edge_index[1].astype(jnp.int32)
    npad = EPAD - E
    srcp = jnp.concatenate([src, jnp.zeros((npad,), jnp.int32)])
    dstp = jnp.concatenate([dst, jnp.full((npad,), N, jnp.int32)])
    src2d = srcp.reshape(EPAD // CHUNK, CHUNK)
    dst2d = dstp.reshape(EPAD // CHUNK, CHUNK)
    lt3 = local_type.astype(jnp.int32).reshape(LGRID, 1, LBLK)
    vt3 = voxel_type.astype(jnp.int32).reshape(GRID, 1, BLK)
    lb = label_hard[0]

    ones_c = jnp.ones((CHUNK,), _f32)
    zeros_1 = jnp.zeros((ZCH,), _f32)
    zeros_16 = jnp.zeros((ZCH, 16), _f32)

    eye8 = jnp.eye(8, dtype=_f32)

    def k8(w):
        return jnp.kron(eye8, w)

    def tile8(v):
        return jnp.tile(v, 8).reshape(1, -1)

    def halves(v, n):
        return [v[16 * i:16 * (i + 1)] for i in range(n)]

    lanes = jnp.arange(128)
    rt16 = (lanes[None, :] // 16 == jnp.arange(8)[:, None]).astype(_f32)
    F = (lanes[:, None] % 16 == lanes[None, :] % 16).astype(_f32) / N

    deg0, deg1 = _deg_sc(dstp, ones_c, zeros_1)
    degT = jnp.stack([deg0, deg1], axis=1)
    dinvf = _kdinv(deg0.reshape(RF, 8), deg1.reshape(RF, 8), rt16)

    t8 = _k1(lt3, local_x, p['mlp_W1'][:128])
    ylo0, yhi0 = _k2(vt3, voxel_x, lb, degT, t8,
                     p['mlp_W1'][128:256], p['mlp_W1'][256:],
                     p['mlp_b1'].reshape(1, HID),
                     p['mlp_W2'], p['mlp_b2'].reshape(1, HID),
                     p['conv0_W'])

    def gnf(li, n):
        return ([tile8(v) for v in halves(p['gn%d_w' % li], n)],
                [tile8(v) for v in halves(p['gn%d_b' % li], n)],
                [tile8(v) for v in halves(p['gn%d_a' % li], n)])

    def wkron(w, n_in, n_out):
        return [k8(w[16 * i:16 * (i + 1), 16 * q:16 * (q + 1)])
                for i in range(n_in) for q in range(n_out)]

    agg0 = _agg_cs16(ylo0, yhi0, src2d, dst2d, zeros_16)
    agg0f = agg0.reshape(2, RF, 128)
    b0 = halves(p['conv0_b'], 2)
    h00, h01, st00, st01 = _k3h2(agg0f, ylo0.reshape(RV, 128),
                                 yhi0.reshape(RV, 128), dinvf,
                                 tile8(b0[0]), tile8(b0[1]))
    y1f = _k4f_21([h00, h01], [st00, st01], gnf(0, 2), F, dinvf,
                  wkron(p['conv1_W'], 2, 1))[0]

    agg1 = _agg_es16(y1f.reshape(NPAD, 16), y1f.reshape(NPAD, 16),
                     src2d, dst2d, zeros_16)
    h1, st1 = _k3h_sum(agg1.reshape(2, RF, 128), y1f, dinvf,
                       tile8(p['conv1_b']))
    y2lof, y2hif = _k4f_12([h1], [st1], gnf(1, 1), F, dinvf,
                           wkron(p['conv2_W'], 1, 2))

    agg2 = _agg_cs16(y2lof.reshape(NPAD, 16), y2hif.reshape(NPAD, 16),
                     src2d, dst2d, zeros_16)
    agg2f = agg2.reshape(2, RF, 128)
    b2 = halves(p['conv2_b'], 2)
    h20, h21, st20, st21 = _k3h2(agg2f, y2lof, y2hif, dinvf,
                                 tile8(b2[0]), tile8(b2[1]))
    qs = _k4f_24([h20, h21], [st20, st21], gnf(2, 2), F, dinvf,
                 wkron(p['conv3_W'], 2, 4))

    agg3a = _agg_cs16(qs[0].reshape(NPAD, 16), qs[1].reshape(NPAD, 16),
                      src2d, dst2d, zeros_16)
    agg3b = _agg_cs16(qs[2].reshape(NPAD, 16), qs[3].reshape(NPAD, 16),
                      src2d, dst2d, zeros_16)
    agg3af = agg3a.reshape(2, RF, 128)
    agg3bf = agg3b.reshape(2, RF, 128)
    b3 = halves(p['conv3_b'], 4)
    h30, h31, st30, st31 = _k3h2(agg3af, qs[0], qs[1], dinvf,
                                 tile8(b3[0]), tile8(b3[1]))
    h32, h33, st32, st33 = _k3h2(agg3bf, qs[2], qs[3], dinvf,
                                 tile8(b3[2]), tile8(b3[3]))
    dws = ([k8(p['dec0_W'][16 * i:16 * (i + 1), :]) for i in range(4)]
           + [tile8(p['dec0_b']), k8(p['dec1_W']), tile8(p['dec1_b']),
              k8(p['dec2_W']), tile8(p['dec2_b']), k8(p['dec3_W']),
              tile8(p['dec3_b'])])
    outf = _k4dec([h30, h31, h32, h33], [st30, st31, st32, st33],
                  gnf(3, 4), F, dws)
    return outf.reshape(NPAD, 1)[:N]
